# Initial kernel scaffold; baseline (speedup 1.0000x reference)
import jax
import jax.numpy as jnp
from jax.experimental import pallas as pl


def kernel(x, noise):
    raise NotImplementedError("write your pallas kernel here")


# speedup vs baseline: 1.8742x; 1.8742x over previous
import functools
import math

import numpy as np
import jax
import jax.numpy as jnp
from jax import lax
from jax.experimental import pallas as pl
from jax.experimental.pallas import tpu as pltpu
from jax.experimental.pallas import tpu_sc as plsc

N = 50176
R = 768
K = 6144
NSEG = 32
ZMAX = 5.5
SEGSCALE = NSEG / (2.0 * ZMAX)
CH = 1568
NCHUNK = N // CH
NW = 32
ROWS_PER_W = R // NW

_zb = np.linspace(-ZMAX, ZMAX, NSEG + 1)
_phi = np.array([0.5 * (1.0 + math.erf(z / math.sqrt(2.0))) for z in _zb])
_yb = _phi * (K - 2) + 1.0
_SLOPE = ((_yb[1:] - _yb[:-1]) / (_zb[1:] - _zb[:-1])).astype(np.float32)
_INTER = (_yb[:-1] - _SLOPE * _zb[:-1]).astype(np.float32)


def _stats_body(x_ref, o_ref):
    xb = x_ref[...]
    s = jnp.sum(xb)
    s2 = jnp.sum(xb * xb)
    col = lax.broadcasted_iota(jnp.int32, (1, 1, 128), 2)
    o_ref[...] = jnp.where(col == 0, s, jnp.where(col == 1, s2, 0.0))


def _gather_vec(v, idx):
    dn = lax.GatherDimensionNumbers(
        offset_dims=(), collapsed_slice_dims=(0,), start_index_map=(0,))
    return lax.gather(v, idx[:, None], dn, (1,),
                      mode=lax.GatherScatterMode.PROMISE_IN_BOUNDS)


def _sc_body(x_hbm, n_hbm, c_hbm, ls_hbm, li_hbm, out_hbm,
             gx, gn, cx, cn, fx, fn, sa, sb, cv, lsv, liv):
    pltpu.sync_copy(c_hbm, cv)
    pltpu.sync_copy(ls_hbm, lsv)
    pltpu.sync_copy(li_hbm, liv)

    iota = lax.iota(jnp.int32, 16)
    mu_x = cv[0, :]
    is_x = cv[1, :]
    mu_n = cv[2, :]
    is_n = cv[3, :]
    av = cv[4, :]
    bv = cv[5, :]

    def bucketize(v, mu, inv_s):
        z = (v - mu) * inv_s
        seg = jnp.clip(((z + ZMAX) * SEGSCALE).astype(jnp.int32), 0, NSEG - 1)
        sl = plsc.load_gather(lsv, [seg])
        ic = plsc.load_gather(liv, [seg])
        return jnp.clip((sl * z + ic).astype(jnp.int32), 0, K - 1)

    def runinfo(ks):
        prev = _gather_vec(ks, jnp.maximum(iota - 1, 0))
        is_new = jnp.logical_or(iota == 0, ks != prev)
        run_start = plsc.cummax(jnp.where(is_new, iota, 0))
        off = iota - run_start
        nxt = _gather_vec(ks, jnp.minimum(iota + 1, 15))
        is_last = jnp.logical_or(iota == 15, ks != nxt)
        return off, is_last

    wid = lax.axis_index("s") * 2 + lax.axis_index("c")

    def row_body(t, carry):
        r = wid * ROWS_PER_W + t

        def zb_(i, _):
            cx[pl.ds(i * 16, 16)] = jnp.zeros((16,), jnp.int32)
            cn[pl.ds(i * 16, 16)] = jnp.zeros((16,), jnp.int32)
            return 0
        lax.fori_loop(0, (K + 16) // 16, zb_, 0)

        def p1_chunk(c, _):
            pltpu.sync_copy(x_hbm.at[pl.ds(r * N + c * CH, CH)], sa)
            pltpu.sync_copy(n_hbm.at[pl.ds(r * N + c * CH, CH)], sb)

            def p1_v(i, _):
                xv = sa[pl.ds(i * 16, 16)]
                nv = sb[pl.ds(i * 16, 16)]
                nsv = av + bv * nv + 0.9 * xv
                px = bucketize(xv, mu_x, is_x)
                pn = bucketize(nsv, mu_n, is_n)
                ksx, _u1 = plsc.sort_key_val(px, px)
                offx, lastx = runinfo(ksx)
                plsc.addupdate_scatter(cx, [ksx], offx + 1, mask=lastx)
                ksn, _u2 = plsc.sort_key_val(pn, pn)
                offn, lastn = runinfo(ksn)
                plsc.addupdate_scatter(cn, [ksn], offn + 1, mask=lastn)
                return 0
            lax.fori_loop(0, CH // 16, p1_v, 0)
            return 0
        lax.fori_loop(0, NCHUNK, p1_chunk, 0)

        def excl(h_ref, f_ref):
            def body(i, carry_v):
                v = h_ref[pl.ds(i * 16, 16)]
                inc = plsc.cumsum(v)
                ex = inc - v + carry_v
                h_ref[pl.ds(i * 16, 16)] = ex
                f_ref[pl.ds(i * 16, 16)] = ex
                return _gather_vec(inc, jnp.full((16,), 15, jnp.int32)) + carry_v
            lax.fori_loop(0, K // 16, body, jnp.zeros((16,), jnp.int32))
            h_ref[pl.ds(K, 16)] = jnp.full((16,), N, jnp.int32)
        excl(cx, fx)
        excl(cn, fn)

        def p2_chunk(c, _):
            pltpu.sync_copy(x_hbm.at[pl.ds(r * N + c * CH, CH)], sa)
            pltpu.sync_copy(n_hbm.at[pl.ds(r * N + c * CH, CH)], sb)

            def p2_v(i, _):
                xv = sa[pl.ds(i * 16, 16)]
                nv = sb[pl.ds(i * 16, 16)]
                nsv = av + bv * nv + 0.9 * xv
                px = bucketize(xv, mu_x, is_x)
                ks, vs = plsc.sort_key_val(px, xv)
                off, last = runinfo(ks)
                base = plsc.load_gather(fx, [ks])
                dest = base + off
                plsc.store_scatter(gx, [dest], vs)
                plsc.store_scatter(fx, [ks], dest + 1, mask=last)
                pn = bucketize(nsv, mu_n, is_n)
                ksn, vsn = plsc.sort_key_val(pn, nsv)
                offn, lastn = runinfo(ksn)
                basen = plsc.load_gather(fn, [ksn])
                destn = basen + offn
                plsc.store_scatter(gn, [destn], vsn)
                plsc.store_scatter(fn, [ksn], destn + 1, mask=lastn)
                return 0
            lax.fori_loop(0, CH // 16, p2_v, 0)
            return 0
        lax.fori_loop(0, NCHUNK, p2_chunk, 0)
        gn[pl.ds(N, 16)] = jnp.full((16,), jnp.inf, jnp.float32)

        def sweep(phase):
            def body(tt, acc):
                idx = (tt * 16 + iota) * 2 + phase
                a = plsc.load_gather(gn, [idx])
                b = plsc.load_gather(gn, [idx + 1])
                sw = a > b
                plsc.store_scatter(gn, [idx], jnp.minimum(a, b))
                plsc.store_scatter(gn, [idx + 1], jnp.maximum(a, b))
                return jnp.logical_or(acc, sw)
            return lax.fori_loop(0, (N // 2) // 16, body,
                                 jnp.zeros((16,), jnp.bool_))

        def w_cond(c):
            return c

        def w_body(c):
            s0 = sweep(0)
            s1 = sweep(1)
            return jnp.any(jnp.logical_or(s0, s1))
        lax.while_loop(w_cond, w_body, jnp.bool_(True))

        def p4_chunk(c, _):
            pltpu.sync_copy(x_hbm.at[pl.ds(r * N + c * CH, CH)], sa)

            def p4_v(i, _):
                xv = sa[pl.ds(i * 16, 16)]
                px = bucketize(xv, mu_x, is_x)
                base = plsc.load_gather(cx, [px])
                end = plsc.load_gather(cx, [px + 1])

                def cc(s):
                    pos, _cnt = s
                    return jnp.any(pos < end)

                def cb(s):
                    pos, cnt = s
                    act = pos < end
                    g = plsc.load_gather(gx, [jnp.minimum(pos, N - 1)])
                    hit = jnp.logical_and(act, g < xv)
                    return pos + 1, cnt + jnp.where(hit, 1, 0)
                _p, rank = lax.while_loop(cc, cb, (base, base))
                sv = plsc.load_gather(gn, [rank])
                sb[pl.ds(i * 16, 16)] = xv + (sv - xv)
                return 0
            lax.fori_loop(0, CH // 16, p4_v, 0)
            pltpu.sync_copy(sb, out_hbm.at[pl.ds(r * N + c * CH, CH)])
            return 0
        lax.fori_loop(0, NCHUNK, p4_chunk, 0)
        return carry
    lax.fori_loop(0, ROWS_PER_W, row_body, 0)


_sc_call = pl.kernel(
    _sc_body,
    mesh=plsc.VectorSubcoreMesh(core_axis_name="c", subcore_axis_name="s"),
    compiler_params=pltpu.CompilerParams(needs_layout_passes=False),
    out_type=jax.ShapeDtypeStruct((R * N,), jnp.float32),
    scratch_types=[
        pltpu.VMEM((N + 16,), jnp.float32),
        pltpu.VMEM((N + 16,), jnp.float32),
        pltpu.VMEM((K + 16,), jnp.int32),
        pltpu.VMEM((K + 16,), jnp.int32),
        pltpu.VMEM((K + 16,), jnp.int32),
        pltpu.VMEM((K + 16,), jnp.int32),
        pltpu.VMEM((CH,), jnp.float32),
        pltpu.VMEM((CH,), jnp.float32),
        pltpu.VMEM((8, 16), jnp.float32),
        pltpu.VMEM((NSEG,), jnp.float32),
        pltpu.VMEM((NSEG,), jnp.float32),
    ],
)


def kernel(x, noise):
    B, C, W, H = x.shape
    x2 = x.reshape(R, N)
    n2 = noise.reshape(R, N)

    part = pl.pallas_call(
        _stats_body,
        grid=(96,),
        in_specs=[pl.BlockSpec((8, N), lambda i: (i, 0))],
        out_specs=pl.BlockSpec((1, 1, 128), lambda i: (i, 0, 0)),
        out_shape=jax.ShapeDtypeStruct((96, 1, 128), jnp.float32),
    )(x2)
    s = jnp.sum(part[:, 0, 0])
    s2 = jnp.sum(part[:, 0, 1])
    m = R * N
    mean = s / m
    var = (s2 - s * s / m) / (m - 1)
    sd = jnp.sqrt(var)

    def bc(v):
        return jnp.broadcast_to(v, (16,)).astype(jnp.float32)
    consts = jnp.stack([
        bc(mean), bc(1.0 / sd), bc(mean), bc(1.0 / (0.9055 * sd)),
        bc(0.1 * mean), bc(0.1 * sd), bc(0.0), bc(0.0)])

    out = _sc_call(x2.reshape(-1), n2.reshape(-1), consts,
                   jnp.asarray(_SLOPE), jnp.asarray(_INTER))
    return out.reshape(B, C, W, H)


# speedup vs baseline: 2.1314x; 1.1372x over previous
import functools
import math

import numpy as np
import jax
import jax.numpy as jnp
from jax import lax
from jax.experimental import pallas as pl
from jax.experimental.pallas import tpu as pltpu
from jax.experimental.pallas import tpu_sc as plsc

N = 50176
R = 768
K = 8192
NSEG = 32
ZMAX = 5.5
SEGSCALE = NSEG / (2.0 * ZMAX)
CH = 1568
NCHUNK = N // CH
NW = 32
ROWS_PER_W = R // NW

_zb = np.linspace(-ZMAX, ZMAX, NSEG + 1)
_phi = np.array([0.5 * (1.0 + math.erf(z / math.sqrt(2.0))) for z in _zb])
_yb = _phi * (K - 2) + 1.0
_SLOPE = ((_yb[1:] - _yb[:-1]) / (_zb[1:] - _zb[:-1])).astype(np.float32)
_INTER = (_yb[:-1] - _SLOPE * _zb[:-1]).astype(np.float32)


def _stats_body(x_ref, o_ref):
    xb = x_ref[...]
    s = jnp.sum(xb)
    s2 = jnp.sum(xb * xb)
    col = lax.broadcasted_iota(jnp.int32, (1, 1, 128), 2)
    o_ref[...] = jnp.where(col == 0, s, jnp.where(col == 1, s2, 0.0))


def _gather_vec(v, idx):
    dn = lax.GatherDimensionNumbers(
        offset_dims=(), collapsed_slice_dims=(0,), start_index_map=(0,))
    return lax.gather(v, idx[:, None], dn, (1,),
                      mode=lax.GatherScatterMode.PROMISE_IN_BOUNDS)


def _sc_body(x_hbm, n_hbm, c_hbm, ls_hbm, li_hbm, out_hbm,
             gx, gn, cx, fx, fn, sa, sb, cv, lsv, liv):
    pltpu.sync_copy(c_hbm, cv)
    pltpu.sync_copy(ls_hbm, lsv)
    pltpu.sync_copy(li_hbm, liv)

    iota = lax.iota(jnp.int32, 16)
    mu_x = cv[0, :]
    is_x = cv[1, :]
    mu_n = cv[2, :]
    is_n = cv[3, :]
    av = cv[4, :]
    bv = cv[5, :]

    def bucketize(v, mu, inv_s):
        z = (v - mu) * inv_s
        seg = jnp.clip(((z + ZMAX) * SEGSCALE).astype(jnp.int32), 0, NSEG - 1)
        sl = plsc.load_gather(lsv, [seg])
        ic = plsc.load_gather(liv, [seg])
        return jnp.clip((sl * z + ic).astype(jnp.int32), 0, K - 1)

    def runinfo(ks):
        prev = _gather_vec(ks, jnp.maximum(iota - 1, 0))
        is_new = jnp.logical_or(iota == 0, ks != prev)
        run_start = plsc.cummax(jnp.where(is_new, iota, 0))
        off = iota - run_start
        nxt = _gather_vec(ks, jnp.minimum(iota + 1, 15))
        is_last = jnp.logical_or(iota == 15, ks != nxt)
        return off, is_last

    wid = lax.axis_index("s") * 2 + lax.axis_index("c")

    def row_body(t, carry):
        r = wid * ROWS_PER_W + t

        def zb_(i, _):
            cx[pl.ds(i * 16, 16)] = jnp.zeros((16,), jnp.int32)
            fn[pl.ds(i * 16, 16)] = jnp.zeros((16,), jnp.int32)
            return 0
        lax.fori_loop(0, (K + 16) // 16, zb_, 0)

        def p1_chunk(c, _):
            pltpu.sync_copy(x_hbm.at[pl.ds(r * N + c * CH, CH)], sa)
            pltpu.sync_copy(n_hbm.at[pl.ds(r * N + c * CH, CH)], sb)

            def p1_v(i, _):
                xv = sa[pl.ds(i * 16, 16)]
                nv = sb[pl.ds(i * 16, 16)]
                nsv = av + bv * nv + 0.9 * xv
                px = bucketize(xv, mu_x, is_x)
                pn = bucketize(nsv, mu_n, is_n)
                ksx, _u1 = plsc.sort_key_val(px, px)
                offx, lastx = runinfo(ksx)
                plsc.addupdate_scatter(cx, [ksx], offx + 1, mask=lastx)
                ksn, _u2 = plsc.sort_key_val(pn, pn)
                offn, lastn = runinfo(ksn)
                plsc.addupdate_scatter(fn, [ksn], offn + 1, mask=lastn)
                return 0
            lax.fori_loop(0, CH // 16, p1_v, 0)
            return 0
        lax.fori_loop(0, NCHUNK, p1_chunk, 0)

        def excl(h_ref, f_ref):
            def body(i, carry_v):
                v = h_ref[pl.ds(i * 16, 16)]
                inc = plsc.cumsum(v)
                ex = inc - v + carry_v
                h_ref[pl.ds(i * 16, 16)] = ex
                if f_ref is not None:
                    f_ref[pl.ds(i * 16, 16)] = ex
                return _gather_vec(inc, jnp.full((16,), 15, jnp.int32)) + carry_v
            lax.fori_loop(0, K // 16, body, jnp.zeros((16,), jnp.int32))
            h_ref[pl.ds(K, 16)] = jnp.full((16,), N, jnp.int32)
        excl(cx, fx)
        excl(fn, None)

        def p2_chunk(c, _):
            pltpu.sync_copy(x_hbm.at[pl.ds(r * N + c * CH, CH)], sa)
            pltpu.sync_copy(n_hbm.at[pl.ds(r * N + c * CH, CH)], sb)

            def p2_v(i, _):
                xv = sa[pl.ds(i * 16, 16)]
                nv = sb[pl.ds(i * 16, 16)]
                nsv = av + bv * nv + 0.9 * xv
                px = bucketize(xv, mu_x, is_x)
                ks, vs = plsc.sort_key_val(px, xv)
                off, last = runinfo(ks)
                base = plsc.load_gather(fx, [ks])
                dest = base + off
                plsc.store_scatter(gx, [dest], vs)
                plsc.store_scatter(fx, [ks], dest + 1, mask=last)
                pn = bucketize(nsv, mu_n, is_n)
                ksn, vsn = plsc.sort_key_val(pn, nsv)
                offn, lastn = runinfo(ksn)
                basen = plsc.load_gather(fn, [ksn])
                destn = basen + offn
                plsc.store_scatter(gn, [destn], vsn)
                plsc.store_scatter(fn, [ksn], destn + 1, mask=lastn)
                return 0
            lax.fori_loop(0, CH // 16, p2_v, 0)
            return 0
        lax.fori_loop(0, NCHUNK, p2_chunk, 0)
        gn[pl.ds(N, 16)] = jnp.full((16,), jnp.inf, jnp.float32)

        def sweep(phase):
            def body(tt, acc):
                idx = (tt * 16 + iota) * 2 + phase
                a = plsc.load_gather(gn, [idx])
                b = plsc.load_gather(gn, [idx + 1])
                sw = a > b
                plsc.store_scatter(gn, [idx], jnp.minimum(a, b))
                plsc.store_scatter(gn, [idx + 1], jnp.maximum(a, b))
                return jnp.logical_or(acc, sw)
            return lax.fori_loop(0, (N // 2) // 16, body,
                                 jnp.zeros((16,), jnp.bool_))

        def w_cond(c):
            return c

        def w_body(c):
            s0 = sweep(0)
            s1 = sweep(1)
            return jnp.any(jnp.logical_or(s0, s1))
        lax.while_loop(w_cond, w_body, jnp.bool_(True))

        def p4_chunk(c, _):
            pltpu.sync_copy(x_hbm.at[pl.ds(r * N + c * CH, CH)], sa)

            def p4_v(i, _):
                xv = sa[pl.ds(i * 16, 16)]
                px = bucketize(xv, mu_x, is_x)
                base = plsc.load_gather(cx, [px])
                end = plsc.load_gather(cx, [px + 1])

                def cc(s):
                    pos, _cnt = s
                    return jnp.any(pos < end)

                def cb(s):
                    pos, cnt = s
                    act = pos < end
                    g = plsc.load_gather(gx, [jnp.minimum(pos, N - 1)])
                    hit = jnp.logical_and(act, g < xv)
                    return pos + 1, cnt + jnp.where(hit, 1, 0)
                _p, rank = lax.while_loop(cc, cb, (base, base))
                sv = plsc.load_gather(gn, [rank])
                sb[pl.ds(i * 16, 16)] = xv + (sv - xv)
                return 0
            lax.fori_loop(0, CH // 16, p4_v, 0)
            pltpu.sync_copy(sb, out_hbm.at[pl.ds(r * N + c * CH, CH)])
            return 0
        lax.fori_loop(0, NCHUNK, p4_chunk, 0)
        return carry
    lax.fori_loop(0, ROWS_PER_W, row_body, 0)


_sc_call = pl.kernel(
    _sc_body,
    mesh=plsc.VectorSubcoreMesh(core_axis_name="c", subcore_axis_name="s"),
    compiler_params=pltpu.CompilerParams(needs_layout_passes=False),
    out_type=jax.ShapeDtypeStruct((R * N,), jnp.float32),
    scratch_types=[
        pltpu.VMEM((N + 16,), jnp.float32),
        pltpu.VMEM((N + 16,), jnp.float32),
        pltpu.VMEM((K + 16,), jnp.int32),
        pltpu.VMEM((K + 16,), jnp.int32),
        pltpu.VMEM((K + 16,), jnp.int32),
        pltpu.VMEM((CH,), jnp.float32),
        pltpu.VMEM((CH,), jnp.float32),
        pltpu.VMEM((8, 16), jnp.float32),
        pltpu.VMEM((NSEG,), jnp.float32),
        pltpu.VMEM((NSEG,), jnp.float32),
    ],
)


def kernel(x, noise):
    B, C, W, H = x.shape
    x2 = x.reshape(R, N)
    n2 = noise.reshape(R, N)

    part = pl.pallas_call(
        _stats_body,
        grid=(96,),
        in_specs=[pl.BlockSpec((8, N), lambda i: (i, 0))],
        out_specs=pl.BlockSpec((1, 1, 128), lambda i: (i, 0, 0)),
        out_shape=jax.ShapeDtypeStruct((96, 1, 128), jnp.float32),
    )(x2)
    s = jnp.sum(part[:, 0, 0])
    s2 = jnp.sum(part[:, 0, 1])
    m = R * N
    mean = s / m
    var = (s2 - s * s / m) / (m - 1)
    sd = jnp.sqrt(var)

    def bc(v):
        return jnp.broadcast_to(v, (16,)).astype(jnp.float32)
    consts = jnp.stack([
        bc(mean), bc(1.0 / sd), bc(mean), bc(1.0 / (0.9055 * sd)),
        bc(0.1 * mean), bc(0.1 * sd), bc(0.0), bc(0.0)])

    out = _sc_call(x2.reshape(-1), n2.reshape(-1), consts,
                   jnp.asarray(_SLOPE), jnp.asarray(_INTER))
    return out.reshape(B, C, W, H)


# speedup vs baseline: 2.2360x; 1.0491x over previous
import functools
import math

import numpy as np
import jax
import jax.numpy as jnp
from jax import lax
from jax.experimental import pallas as pl
from jax.experimental.pallas import tpu as pltpu
from jax.experimental.pallas import tpu_sc as plsc

N = 50176
R = 768
K = 8192
NSEG = 32
ZMAX = 5.5
SEGSCALE = NSEG / (2.0 * ZMAX)
CH = 1568
NCHUNK = N // CH
NW = 32
ROWS_PER_W = R // NW

_zb = np.linspace(-ZMAX, ZMAX, NSEG + 1)
_phi = np.array([0.5 * (1.0 + math.erf(z / math.sqrt(2.0))) for z in _zb])
_yb = _phi * (K - 2) + 1.0
_SLOPE = ((_yb[1:] - _yb[:-1]) / (_zb[1:] - _zb[:-1])).astype(np.float32)
_INTER = (_yb[:-1] - _SLOPE * _zb[:-1]).astype(np.float32)


def _stats_body(x_ref, o_ref):
    xb = x_ref[...]
    s = jnp.sum(xb)
    s2 = jnp.sum(xb * xb)
    col = lax.broadcasted_iota(jnp.int32, (1, 1, 128), 2)
    o_ref[...] = jnp.where(col == 0, s, jnp.where(col == 1, s2, 0.0))


def _gather_vec(v, idx):
    dn = lax.GatherDimensionNumbers(
        offset_dims=(), collapsed_slice_dims=(0,), start_index_map=(0,))
    return lax.gather(v, idx[:, None], dn, (1,),
                      mode=lax.GatherScatterMode.PROMISE_IN_BOUNDS)


def _sc_body(x_hbm, n_hbm, c_hbm, ls_hbm, li_hbm, out_hbm,
             gx, gn, cx, fx, fn, sa, sb, cv, lsv, liv):
    pltpu.sync_copy(c_hbm, cv)
    pltpu.sync_copy(ls_hbm, lsv)
    pltpu.sync_copy(li_hbm, liv)

    iota = lax.iota(jnp.int32, 16)
    mu_x = cv[0, :]
    is_x = cv[1, :]
    mu_n = cv[2, :]
    is_n = cv[3, :]
    av = cv[4, :]
    bv = cv[5, :]

    def bucketize(v, mu, inv_s):
        z = (v - mu) * inv_s
        seg = jnp.clip(((z + ZMAX) * SEGSCALE).astype(jnp.int32), 0, NSEG - 1)
        sl = plsc.load_gather(lsv, [seg])
        ic = plsc.load_gather(liv, [seg])
        return jnp.clip((sl * z + ic).astype(jnp.int32), 0, K - 1)

    def runinfo(ks):
        prev = _gather_vec(ks, jnp.maximum(iota - 1, 0))
        is_new = jnp.logical_or(iota == 0, ks != prev)
        run_start = plsc.cummax(jnp.where(is_new, iota, 0))
        off = iota - run_start
        nxt = _gather_vec(ks, jnp.minimum(iota + 1, 15))
        is_last = jnp.logical_or(iota == 15, ks != nxt)
        return off, is_last

    wid = lax.axis_index("s") * 2 + lax.axis_index("c")

    def row_body(t, carry):
        r = wid * ROWS_PER_W + t

        def zb_(i, _):
            cx[pl.ds(i * 16, 16)] = jnp.zeros((16,), jnp.int32)
            fn[pl.ds(i * 16, 16)] = jnp.zeros((16,), jnp.int32)
            return 0
        lax.fori_loop(0, (K + 16) // 16, zb_, 0)

        def p1_chunk(c, _):
            pltpu.sync_copy(x_hbm.at[pl.ds(r * N + c * CH, CH)], sa)
            pltpu.sync_copy(n_hbm.at[pl.ds(r * N + c * CH, CH)], sb)

            def p1_v(i, _):
                xv = sa[pl.ds(i * 16, 16)]
                nv = sb[pl.ds(i * 16, 16)]
                nsv = av + bv * nv + 0.9 * xv
                px = bucketize(xv, mu_x, is_x)
                pn = bucketize(nsv, mu_n, is_n)
                ksx, _u1 = plsc.sort_key_val(px, px)
                offx, lastx = runinfo(ksx)
                plsc.addupdate_scatter(cx, [ksx], offx + 1, mask=lastx)
                ksn, _u2 = plsc.sort_key_val(pn, pn)
                offn, lastn = runinfo(ksn)
                plsc.addupdate_scatter(fn, [ksn], offn + 1, mask=lastn)
                return 0
            lax.fori_loop(0, CH // 16, p1_v, 0)
            return 0
        lax.fori_loop(0, NCHUNK, p1_chunk, 0)

        def excl(h_ref, f_ref):
            def body(i, carry_v):
                v = h_ref[pl.ds(i * 16, 16)]
                inc = plsc.cumsum(v)
                ex = inc - v + carry_v
                h_ref[pl.ds(i * 16, 16)] = ex
                if f_ref is not None:
                    f_ref[pl.ds(i * 16, 16)] = ex
                return _gather_vec(inc, jnp.full((16,), 15, jnp.int32)) + carry_v
            lax.fori_loop(0, K // 16, body, jnp.zeros((16,), jnp.int32))
            h_ref[pl.ds(K, 16)] = jnp.full((16,), N, jnp.int32)
        excl(cx, fx)
        excl(fn, None)

        def p2_chunk(c, _):
            pltpu.sync_copy(x_hbm.at[pl.ds(r * N + c * CH, CH)], sa)
            pltpu.sync_copy(n_hbm.at[pl.ds(r * N + c * CH, CH)], sb)

            def p2_v(i, _):
                xv = sa[pl.ds(i * 16, 16)]
                nv = sb[pl.ds(i * 16, 16)]
                nsv = av + bv * nv + 0.9 * xv
                px = bucketize(xv, mu_x, is_x)
                ks, vs = plsc.sort_key_val(px, xv)
                off, last = runinfo(ks)
                base = plsc.load_gather(fx, [ks])
                dest = base + off
                plsc.store_scatter(gx, [dest], vs)
                plsc.store_scatter(fx, [ks], dest + 1, mask=last)
                pn = bucketize(nsv, mu_n, is_n)
                ksn, vsn = plsc.sort_key_val(pn, nsv)
                offn, lastn = runinfo(ksn)
                basen = plsc.load_gather(fn, [ksn])
                destn = basen + offn
                plsc.store_scatter(gn, [destn], vsn)
                plsc.store_scatter(fn, [ksn], destn + 1, mask=lastn)
                return 0
            lax.fori_loop(0, CH // 16, p2_v, 0)
            return 0
        lax.fori_loop(0, NCHUNK, p2_chunk, 0)
        gn[pl.ds(N, 16)] = jnp.full((16,), jnp.inf, jnp.float32)

        def blksort(i, _):
            v = gn[pl.ds(i * 16, 16)]
            sv, _u = plsc.sort_key_val(v, v)
            gn[pl.ds(i * 16, 16)] = sv
            return 0
        lax.fori_loop(0, N // 16, blksort, 0)

        def merge_sweep(phase):
            def body(tt, acc):
                base = (tt * 2 + phase) * 16
                a = gn[pl.ds(base, 16)]
                b = gn[pl.ds(base + 16, 16)]
                rb = lax.rev(b, (0,))
                lo = jnp.minimum(a, rb)
                hi = jnp.maximum(a, rb)
                slo, _u1 = plsc.sort_key_val(lo, lo)
                shi, _u2 = plsc.sort_key_val(hi, hi)
                gn[pl.ds(base, 16)] = slo
                gn[pl.ds(base + 16, 16)] = shi
                ch = jnp.logical_or(slo != a, shi != b)
                return jnp.logical_or(acc, ch)
            return lax.fori_loop(0, N // 32, body, jnp.zeros((16,), jnp.bool_))

        def w_cond(c):
            return c

        def w_body(c):
            s0 = merge_sweep(0)
            s1 = merge_sweep(1)
            return jnp.any(jnp.logical_or(s0, s1))
        lax.while_loop(w_cond, w_body, jnp.bool_(True))

        def p4_chunk(c, _):
            pltpu.sync_copy(x_hbm.at[pl.ds(r * N + c * CH, CH)], sa)

            def p4_v(i, _):
                xv = sa[pl.ds(i * 16, 16)]
                px = bucketize(xv, mu_x, is_x)
                base = plsc.load_gather(cx, [px])
                end = plsc.load_gather(cx, [px + 1])

                def cc(s):
                    pos, _cnt = s
                    return jnp.any(pos < end)

                def cb(s):
                    pos, cnt = s
                    act = pos < end
                    g = plsc.load_gather(gx, [jnp.minimum(pos, N - 1)])
                    hit = jnp.logical_and(act, g < xv)
                    return pos + 1, cnt + jnp.where(hit, 1, 0)
                _p, rank = lax.while_loop(cc, cb, (base, base))
                sv = plsc.load_gather(gn, [rank])
                sb[pl.ds(i * 16, 16)] = xv + (sv - xv)
                return 0
            lax.fori_loop(0, CH // 16, p4_v, 0)
            pltpu.sync_copy(sb, out_hbm.at[pl.ds(r * N + c * CH, CH)])
            return 0
        lax.fori_loop(0, NCHUNK, p4_chunk, 0)
        return carry
    lax.fori_loop(0, ROWS_PER_W, row_body, 0)


_sc_call = pl.kernel(
    _sc_body,
    mesh=plsc.VectorSubcoreMesh(core_axis_name="c", subcore_axis_name="s"),
    compiler_params=pltpu.CompilerParams(needs_layout_passes=False),
    out_type=jax.ShapeDtypeStruct((R * N,), jnp.float32),
    scratch_types=[
        pltpu.VMEM((N + 16,), jnp.float32),
        pltpu.VMEM((N + 16,), jnp.float32),
        pltpu.VMEM((K + 16,), jnp.int32),
        pltpu.VMEM((K + 16,), jnp.int32),
        pltpu.VMEM((K + 16,), jnp.int32),
        pltpu.VMEM((CH,), jnp.float32),
        pltpu.VMEM((CH,), jnp.float32),
        pltpu.VMEM((8, 16), jnp.float32),
        pltpu.VMEM((NSEG,), jnp.float32),
        pltpu.VMEM((NSEG,), jnp.float32),
    ],
)


def kernel(x, noise):
    B, C, W, H = x.shape
    x2 = x.reshape(R, N)
    n2 = noise.reshape(R, N)

    part = pl.pallas_call(
        _stats_body,
        grid=(96,),
        in_specs=[pl.BlockSpec((8, N), lambda i: (i, 0))],
        out_specs=pl.BlockSpec((1, 1, 128), lambda i: (i, 0, 0)),
        out_shape=jax.ShapeDtypeStruct((96, 1, 128), jnp.float32),
    )(x2)
    s = jnp.sum(part[:, 0, 0])
    s2 = jnp.sum(part[:, 0, 1])
    m = R * N
    mean = s / m
    var = (s2 - s * s / m) / (m - 1)
    sd = jnp.sqrt(var)

    def bc(v):
        return jnp.broadcast_to(v, (16,)).astype(jnp.float32)
    consts = jnp.stack([
        bc(mean), bc(1.0 / sd), bc(mean), bc(1.0 / (0.9055 * sd)),
        bc(0.1 * mean), bc(0.1 * sd), bc(0.0), bc(0.0)])

    out = _sc_call(x2.reshape(-1), n2.reshape(-1), consts,
                   jnp.asarray(_SLOPE), jnp.asarray(_INTER))
    return out.reshape(B, C, W, H)


# speedup vs baseline: 2.2360x; 1.0000x over previous
import functools
import math

import numpy as np
import jax
import jax.numpy as jnp
from jax import lax
from jax.experimental import pallas as pl
from jax.experimental.pallas import tpu as pltpu
from jax.experimental.pallas import tpu_sc as plsc

N = 50176
R = 768
K = 8192
NSEG = 32
ZMAX = 5.5
SEGSCALE = NSEG / (2.0 * ZMAX)
CH = 1568
NCHUNK = N // CH
NW = 32
ROWS_PER_W = R // NW

_zb = np.linspace(-ZMAX, ZMAX, NSEG + 1)
_phi = np.array([0.5 * (1.0 + math.erf(z / math.sqrt(2.0))) for z in _zb])
_yb = _phi * (K - 2) + 1.0
_SLOPE = ((_yb[1:] - _yb[:-1]) / (_zb[1:] - _zb[:-1])).astype(np.float32)
_INTER = (_yb[:-1] - _SLOPE * _zb[:-1]).astype(np.float32)


def _stats_body(x_ref, o_ref):
    xb = x_ref[...]
    s = jnp.sum(xb)
    s2 = jnp.sum(xb * xb)
    col = lax.broadcasted_iota(jnp.int32, (1, 1, 128), 2)
    o_ref[...] = jnp.where(col == 0, s, jnp.where(col == 1, s2, 0.0))


def _gather_vec(v, idx):
    dn = lax.GatherDimensionNumbers(
        offset_dims=(), collapsed_slice_dims=(0,), start_index_map=(0,))
    return lax.gather(v, idx[:, None], dn, (1,),
                      mode=lax.GatherScatterMode.PROMISE_IN_BOUNDS)


def _sc_body(x_hbm, n_hbm, c_hbm, ls_hbm, li_hbm, out_hbm,
             gx, gn, cx, fx, fn, sa, sb, cv, lsv, liv):
    pltpu.sync_copy(c_hbm, cv)
    pltpu.sync_copy(ls_hbm, lsv)
    pltpu.sync_copy(li_hbm, liv)

    iota = lax.iota(jnp.int32, 16)
    mu_x = cv[0, :]
    is_x = cv[1, :]
    mu_n = cv[2, :]
    is_n = cv[3, :]
    av = cv[4, :]
    bv = cv[5, :]

    def bucketize(v, mu, inv_s):
        z = (v - mu) * inv_s
        seg = jnp.clip(((z + ZMAX) * SEGSCALE).astype(jnp.int32), 0, NSEG - 1)
        sl = plsc.load_gather(lsv, [seg])
        ic = plsc.load_gather(liv, [seg])
        return jnp.clip((sl * z + ic).astype(jnp.int32), 0, K - 1)

    def runinfo(ks):
        prev = _gather_vec(ks, jnp.maximum(iota - 1, 0))
        is_new = jnp.logical_or(iota == 0, ks != prev)
        run_start = plsc.cummax(jnp.where(is_new, iota, 0))
        off = iota - run_start
        nxt = _gather_vec(ks, jnp.minimum(iota + 1, 15))
        is_last = jnp.logical_or(iota == 15, ks != nxt)
        return off, is_last

    wid = lax.axis_index("s") * 2 + lax.axis_index("c")

    def row_body(t, carry):
        r = wid * ROWS_PER_W + t

        def zb_(i, _):
            cx[pl.ds(i * 16, 16)] = jnp.zeros((16,), jnp.int32)
            fn[pl.ds(i * 16, 16)] = jnp.zeros((16,), jnp.int32)
            return 0
        with jax.named_scope("ph_zero"):
            lax.fori_loop(0, (K + 16) // 16, zb_, 0)

        def p1_chunk(c, _):
            pltpu.sync_copy(x_hbm.at[pl.ds(r * N + c * CH, CH)], sa)
            pltpu.sync_copy(n_hbm.at[pl.ds(r * N + c * CH, CH)], sb)

            def p1_v(i, _):
                xv = sa[pl.ds(i * 16, 16)]
                nv = sb[pl.ds(i * 16, 16)]
                nsv = av + bv * nv + 0.9 * xv
                px = bucketize(xv, mu_x, is_x)
                pn = bucketize(nsv, mu_n, is_n)
                ksx, _u1 = plsc.sort_key_val(px, px)
                offx, lastx = runinfo(ksx)
                plsc.addupdate_scatter(cx, [ksx], offx + 1, mask=lastx)
                ksn, _u2 = plsc.sort_key_val(pn, pn)
                offn, lastn = runinfo(ksn)
                plsc.addupdate_scatter(fn, [ksn], offn + 1, mask=lastn)
                return 0
            lax.fori_loop(0, CH // 16, p1_v, 0)
            return 0
        with jax.named_scope("ph_hist"):
            lax.fori_loop(0, NCHUNK, p1_chunk, 0)

        def excl(h_ref, f_ref):
            def body(i, carry_v):
                v = h_ref[pl.ds(i * 16, 16)]
                inc = plsc.cumsum(v)
                ex = inc - v + carry_v
                h_ref[pl.ds(i * 16, 16)] = ex
                if f_ref is not None:
                    f_ref[pl.ds(i * 16, 16)] = ex
                return _gather_vec(inc, jnp.full((16,), 15, jnp.int32)) + carry_v
            lax.fori_loop(0, K // 16, body, jnp.zeros((16,), jnp.int32))
            h_ref[pl.ds(K, 16)] = jnp.full((16,), N, jnp.int32)
        with jax.named_scope("ph_cumsum"):
            excl(cx, fx)
            excl(fn, None)

        def p2_chunk(c, _):
            pltpu.sync_copy(x_hbm.at[pl.ds(r * N + c * CH, CH)], sa)
            pltpu.sync_copy(n_hbm.at[pl.ds(r * N + c * CH, CH)], sb)

            def p2_v(i, _):
                xv = sa[pl.ds(i * 16, 16)]
                nv = sb[pl.ds(i * 16, 16)]
                nsv = av + bv * nv + 0.9 * xv
                px = bucketize(xv, mu_x, is_x)
                ks, vs = plsc.sort_key_val(px, xv)
                off, last = runinfo(ks)
                base = plsc.load_gather(fx, [ks])
                dest = base + off
                plsc.store_scatter(gx, [dest], vs)
                plsc.store_scatter(fx, [ks], dest + 1, mask=last)
                pn = bucketize(nsv, mu_n, is_n)
                ksn, vsn = plsc.sort_key_val(pn, nsv)
                offn, lastn = runinfo(ksn)
                basen = plsc.load_gather(fn, [ksn])
                destn = basen + offn
                plsc.store_scatter(gn, [destn], vsn)
                plsc.store_scatter(fn, [ksn], destn + 1, mask=lastn)
                return 0
            lax.fori_loop(0, CH // 16, p2_v, 0)
            return 0
        with jax.named_scope("ph_scatter"):
            lax.fori_loop(0, NCHUNK, p2_chunk, 0)
        gn[pl.ds(N, 16)] = jnp.full((16,), jnp.inf, jnp.float32)

        def blksort(i, _):
            v = gn[pl.ds(i * 16, 16)]
            sv, _u = plsc.sort_key_val(v, v)
            gn[pl.ds(i * 16, 16)] = sv
            return 0
        with jax.named_scope("ph_blksort"):
            lax.fori_loop(0, N // 16, blksort, 0)

        def merge_sweep(phase):
            def body(tt, acc):
                base = (tt * 2 + phase) * 16
                a = gn[pl.ds(base, 16)]
                b = gn[pl.ds(base + 16, 16)]
                rb = lax.rev(b, (0,))
                lo = jnp.minimum(a, rb)
                hi = jnp.maximum(a, rb)
                slo, _u1 = plsc.sort_key_val(lo, lo)
                shi, _u2 = plsc.sort_key_val(hi, hi)
                gn[pl.ds(base, 16)] = slo
                gn[pl.ds(base + 16, 16)] = shi
                ch = jnp.logical_or(slo != a, shi != b)
                return jnp.logical_or(acc, ch)
            return lax.fori_loop(0, N // 32, body, jnp.zeros((16,), jnp.bool_))

        def w_cond(c):
            return c

        def w_body(c):
            s0 = merge_sweep(0)
            s1 = merge_sweep(1)
            return jnp.any(jnp.logical_or(s0, s1))
        with jax.named_scope("ph_merge"):
            lax.while_loop(w_cond, w_body, jnp.bool_(True))

        def p4_chunk(c, _):
            pltpu.sync_copy(x_hbm.at[pl.ds(r * N + c * CH, CH)], sa)

            def p4_v(i, _):
                xv = sa[pl.ds(i * 16, 16)]
                px = bucketize(xv, mu_x, is_x)
                base = plsc.load_gather(cx, [px])
                end = plsc.load_gather(cx, [px + 1])

                def cc(s):
                    pos, _cnt = s
                    return jnp.any(pos < end)

                def cb(s):
                    pos, cnt = s
                    act = pos < end
                    g = plsc.load_gather(gx, [jnp.minimum(pos, N - 1)])
                    hit = jnp.logical_and(act, g < xv)
                    return pos + 1, cnt + jnp.where(hit, 1, 0)
                _p, rank = lax.while_loop(cc, cb, (base, base))
                sv = plsc.load_gather(gn, [rank])
                sb[pl.ds(i * 16, 16)] = xv + (sv - xv)
                return 0
            lax.fori_loop(0, CH // 16, p4_v, 0)
            pltpu.sync_copy(sb, out_hbm.at[pl.ds(r * N + c * CH, CH)])
            return 0
        with jax.named_scope("ph_rank"):
            lax.fori_loop(0, NCHUNK, p4_chunk, 0)
        return carry
    lax.fori_loop(0, ROWS_PER_W, row_body, 0)


_sc_call = pl.kernel(
    _sc_body,
    mesh=plsc.VectorSubcoreMesh(core_axis_name="c", subcore_axis_name="s"),
    compiler_params=pltpu.CompilerParams(needs_layout_passes=False),
    out_type=jax.ShapeDtypeStruct((R * N,), jnp.float32),
    scratch_types=[
        pltpu.VMEM((N + 16,), jnp.float32),
        pltpu.VMEM((N + 16,), jnp.float32),
        pltpu.VMEM((K + 16,), jnp.int32),
        pltpu.VMEM((K + 16,), jnp.int32),
        pltpu.VMEM((K + 16,), jnp.int32),
        pltpu.VMEM((CH,), jnp.float32),
        pltpu.VMEM((CH,), jnp.float32),
        pltpu.VMEM((8, 16), jnp.float32),
        pltpu.VMEM((NSEG,), jnp.float32),
        pltpu.VMEM((NSEG,), jnp.float32),
    ],
)


def kernel(x, noise):
    B, C, W, H = x.shape
    x2 = x.reshape(R, N)
    n2 = noise.reshape(R, N)

    part = pl.pallas_call(
        _stats_body,
        grid=(96,),
        in_specs=[pl.BlockSpec((8, N), lambda i: (i, 0))],
        out_specs=pl.BlockSpec((1, 1, 128), lambda i: (i, 0, 0)),
        out_shape=jax.ShapeDtypeStruct((96, 1, 128), jnp.float32),
    )(x2)
    s = jnp.sum(part[:, 0, 0])
    s2 = jnp.sum(part[:, 0, 1])
    m = R * N
    mean = s / m
    var = (s2 - s * s / m) / (m - 1)
    sd = jnp.sqrt(var)

    def bc(v):
        return jnp.broadcast_to(v, (16,)).astype(jnp.float32)
    consts = jnp.stack([
        bc(mean), bc(1.0 / sd), bc(mean), bc(1.0 / (0.9055 * sd)),
        bc(0.1 * mean), bc(0.1 * sd), bc(0.0), bc(0.0)])

    out = _sc_call(x2.reshape(-1), n2.reshape(-1), consts,
                   jnp.asarray(_SLOPE), jnp.asarray(_INTER))
    return out.reshape(B, C, W, H)


# speedup vs baseline: 3.4048x; 1.5227x over previous
import functools
import math

import numpy as np
import jax
import jax.numpy as jnp
from jax import lax
from jax.experimental import pallas as pl
from jax.experimental.pallas import tpu as pltpu
from jax.experimental.pallas import tpu_sc as plsc

N = 50176
R = 768
K = 8192
NSEG = 32
ZMAX = 5.5
SEGSCALE = NSEG / (2.0 * ZMAX)
CH = 1792
NCHUNK = N // CH
NW = 32
ROWS_PER_W = R // NW

_zb = np.linspace(-ZMAX, ZMAX, NSEG + 1)
_phi = np.array([0.5 * (1.0 + math.erf(z / math.sqrt(2.0))) for z in _zb])
_yb = _phi * (K - 2) + 1.0
_SLOPE = ((_yb[1:] - _yb[:-1]) / (_zb[1:] - _zb[:-1])).astype(np.float32)
_INTER = (_yb[:-1] - _SLOPE * _zb[:-1]).astype(np.float32)


def _stats_body(x_ref, o_ref):
    xb = x_ref[...]
    s = jnp.sum(xb)
    s2 = jnp.sum(xb * xb)
    col = lax.broadcasted_iota(jnp.int32, (1, 1, 128), 2)
    o_ref[...] = jnp.where(col == 0, s, jnp.where(col == 1, s2, 0.0))


def _gather_vec(v, idx):
    dn = lax.GatherDimensionNumbers(
        offset_dims=(), collapsed_slice_dims=(0,), start_index_map=(0,))
    return lax.gather(v, idx[:, None], dn, (1,),
                      mode=lax.GatherScatterMode.PROMISE_IN_BOUNDS)


def _sc_body(x_hbm, n_hbm, c_hbm, ls_hbm, li_hbm, out_hbm,
             gx, gn, cx, fx, fn, sa, sb, cv, lsv, liv):
    pltpu.sync_copy(c_hbm, cv)
    pltpu.sync_copy(ls_hbm, lsv)
    pltpu.sync_copy(li_hbm, liv)

    iota = lax.iota(jnp.int32, 16)
    mu_x = cv[0, :]
    is_x = cv[1, :]
    mu_n = cv[2, :]
    is_n = cv[3, :]
    av = cv[4, :]
    bv = cv[5, :]

    def bucketize(v, mu, inv_s):
        z = (v - mu) * inv_s
        seg = jnp.clip(((z + ZMAX) * SEGSCALE).astype(jnp.int32), 0, NSEG - 1)
        sl = plsc.load_gather(lsv, [seg])
        ic = plsc.load_gather(liv, [seg])
        return jnp.clip((sl * z + ic).astype(jnp.int32), 0, K - 1)

    wid = lax.axis_index("s") * 2 + lax.axis_index("c")

    def row_body(t, carry):
        r = wid * ROWS_PER_W + t

        def zb_(i, _):
            cx[pl.ds(i * 16, 16)] = jnp.zeros((16,), jnp.int32)
            fn[pl.ds(i * 16, 16)] = jnp.zeros((16,), jnp.int32)
            return 0
        with jax.named_scope("ph_zero"):
            lax.fori_loop(0, (K + 16) // 16, zb_, 0)

        def p1_chunk(c, _):
            pltpu.sync_copy(x_hbm.at[pl.ds(r * N + c * CH, CH)], sa)
            pltpu.sync_copy(n_hbm.at[pl.ds(r * N + c * CH, CH)], sb)

            def p1_v(ii, _):
                for u in range(4):
                    i = ii * 4 + u
                    xv = sa[pl.ds(i * 16, 16)]
                    nv = sb[pl.ds(i * 16, 16)]
                    nsv = av + bv * nv + 0.9 * xv
                    px = bucketize(xv, mu_x, is_x)
                    pn = bucketize(nsv, mu_n, is_n)
                    cntx, lastx = plsc.scan_count(px)
                    plsc.addupdate_scatter(cx, [px], cntx, mask=lastx)
                    cntn, lastn = plsc.scan_count(pn)
                    plsc.addupdate_scatter(fn, [pn], cntn, mask=lastn)
                return 0
            lax.fori_loop(0, CH // 64, p1_v, 0)
            return 0
        with jax.named_scope("ph_hist"):
            lax.fori_loop(0, NCHUNK, p1_chunk, 0)

        def excl(h_ref, f_ref):
            def body(i, carry_v):
                v = h_ref[pl.ds(i * 16, 16)]
                inc = plsc.cumsum(v)
                ex = inc - v + carry_v
                h_ref[pl.ds(i * 16, 16)] = ex
                if f_ref is not None:
                    f_ref[pl.ds(i * 16, 16)] = ex
                return _gather_vec(inc, jnp.full((16,), 15, jnp.int32)) + carry_v
            lax.fori_loop(0, K // 16, body, jnp.zeros((16,), jnp.int32))
            h_ref[pl.ds(K, 16)] = jnp.full((16,), N, jnp.int32)
        with jax.named_scope("ph_cumsum"):
            excl(cx, fx)
            excl(fn, None)

        def p2_chunk(c, _):
            pltpu.sync_copy(x_hbm.at[pl.ds(r * N + c * CH, CH)], sa)
            pltpu.sync_copy(n_hbm.at[pl.ds(r * N + c * CH, CH)], sb)

            def p2_v(ii, _):
                for u in range(4):
                    i = ii * 4 + u
                    xv = sa[pl.ds(i * 16, 16)]
                    nv = sb[pl.ds(i * 16, 16)]
                    nsv = av + bv * nv + 0.9 * xv
                    px = bucketize(xv, mu_x, is_x)
                    cnt, last = plsc.scan_count(px)
                    base = plsc.load_gather(fx, [px])
                    dest = base + cnt - 1
                    plsc.store_scatter(gx, [dest], xv)
                    plsc.store_scatter(fx, [px], base + cnt, mask=last)
                    pn = bucketize(nsv, mu_n, is_n)
                    cntn, lastn = plsc.scan_count(pn)
                    basen = plsc.load_gather(fn, [pn])
                    destn = basen + cntn - 1
                    plsc.store_scatter(gn, [destn], nsv)
                    plsc.store_scatter(fn, [pn], basen + cntn, mask=lastn)
                return 0
            lax.fori_loop(0, CH // 64, p2_v, 0)
            return 0
        with jax.named_scope("ph_scatter"):
            lax.fori_loop(0, NCHUNK, p2_chunk, 0)
        gn[pl.ds(N, 16)] = jnp.full((16,), jnp.inf, jnp.float32)

        def blksort(ii, _):
            for u in range(4):
                i = ii * 4 + u
                v = gn[pl.ds(i * 16, 16)]
                sv, _u = plsc.sort_key_val(v, v)
                gn[pl.ds(i * 16, 16)] = sv
            return 0
        with jax.named_scope("ph_blksort"):
            lax.fori_loop(0, N // 64, blksort, 0)

        def merge_sweep(phase):
            def body(tt, acc):
                for u in range(2):
                    t = tt * 2 + u
                    base = (t * 2 + phase) * 16
                    a = gn[pl.ds(base, 16)]
                    b = gn[pl.ds(base + 16, 16)]
                    rb = lax.rev(b, (0,))
                    lo = jnp.minimum(a, rb)
                    hi = jnp.maximum(a, rb)
                    slo, _u1 = plsc.sort_key_val(lo, lo)
                    shi, _u2 = plsc.sort_key_val(hi, hi)
                    gn[pl.ds(base, 16)] = slo
                    gn[pl.ds(base + 16, 16)] = shi
                    ch = jnp.logical_or(slo != a, shi != b)
                    acc = jnp.logical_or(acc, ch)
                return acc
            return lax.fori_loop(0, N // 64, body, jnp.zeros((16,), jnp.bool_))

        def w_cond(c):
            return c

        def w_body(c):
            s0 = merge_sweep(0)
            s1 = merge_sweep(1)
            return jnp.any(jnp.logical_or(s0, s1))
        with jax.named_scope("ph_merge"):
            lax.while_loop(w_cond, w_body, jnp.bool_(True))

        def p4_chunk(c, _):
            pltpu.sync_copy(x_hbm.at[pl.ds(r * N + c * CH, CH)], sa)

            def p4_v(i, _):
                xv = sa[pl.ds(i * 16, 16)]
                px = bucketize(xv, mu_x, is_x)
                base = plsc.load_gather(cx, [px])
                end = plsc.load_gather(cx, [px + 1])

                def cc(s):
                    pos, _cnt = s
                    return jnp.any(pos < end)

                def cb(s):
                    pos, cnt = s
                    act0 = pos < end
                    g0 = plsc.load_gather(gx, [jnp.minimum(pos, N - 1)])
                    hit0 = jnp.logical_and(act0, g0 < xv)
                    act1 = pos + 1 < end
                    g1 = plsc.load_gather(gx, [jnp.minimum(pos + 1, N - 1)])
                    hit1 = jnp.logical_and(act1, g1 < xv)
                    cnt = cnt + jnp.where(hit0, 1, 0) + jnp.where(hit1, 1, 0)
                    return pos + 2, cnt
                _p, rank = lax.while_loop(cc, cb, (base, base))
                sv = plsc.load_gather(gn, [rank])
                sb[pl.ds(i * 16, 16)] = xv + (sv - xv)
                return 0
            lax.fori_loop(0, CH // 16, p4_v, 0)
            pltpu.sync_copy(sb, out_hbm.at[pl.ds(r * N + c * CH, CH)])
            return 0
        with jax.named_scope("ph_rank"):
            lax.fori_loop(0, NCHUNK, p4_chunk, 0)
        return carry
    lax.fori_loop(0, ROWS_PER_W, row_body, 0)


_sc_call = pl.kernel(
    _sc_body,
    mesh=plsc.VectorSubcoreMesh(core_axis_name="c", subcore_axis_name="s"),
    compiler_params=pltpu.CompilerParams(needs_layout_passes=False),
    out_type=jax.ShapeDtypeStruct((R * N,), jnp.float32),
    scratch_types=[
        pltpu.VMEM((N + 16,), jnp.float32),
        pltpu.VMEM((N + 16,), jnp.float32),
        pltpu.VMEM((K + 16,), jnp.int32),
        pltpu.VMEM((K + 16,), jnp.int32),
        pltpu.VMEM((K + 16,), jnp.int32),
        pltpu.VMEM((CH,), jnp.float32),
        pltpu.VMEM((CH,), jnp.float32),
        pltpu.VMEM((8, 16), jnp.float32),
        pltpu.VMEM((NSEG,), jnp.float32),
        pltpu.VMEM((NSEG,), jnp.float32),
    ],
)


def kernel(x, noise):
    B, C, W, H = x.shape
    x2 = x.reshape(R, N)
    n2 = noise.reshape(R, N)

    part = pl.pallas_call(
        _stats_body,
        grid=(96,),
        in_specs=[pl.BlockSpec((8, N), lambda i: (i, 0))],
        out_specs=pl.BlockSpec((1, 1, 128), lambda i: (i, 0, 0)),
        out_shape=jax.ShapeDtypeStruct((96, 1, 128), jnp.float32),
    )(x2)
    s = jnp.sum(part[:, 0, 0])
    s2 = jnp.sum(part[:, 0, 1])
    m = R * N
    mean = s / m
    var = (s2 - s * s / m) / (m - 1)
    sd = jnp.sqrt(var)

    def bc(v):
        return jnp.broadcast_to(v, (16,)).astype(jnp.float32)
    consts = jnp.stack([
        bc(mean), bc(1.0 / sd), bc(mean), bc(1.0 / (0.9055 * sd)),
        bc(0.1 * mean), bc(0.1 * sd), bc(0.0), bc(0.0)])

    out = _sc_call(x2.reshape(-1), n2.reshape(-1), consts,
                   jnp.asarray(_SLOPE), jnp.asarray(_INTER))
    return out.reshape(B, C, W, H)


# speedup vs baseline: 5.9748x; 1.7548x over previous
import math

import numpy as np
import jax
import jax.numpy as jnp
from jax import lax
from jax.experimental import pallas as pl
from jax.experimental.pallas import tpu as pltpu
from jax.experimental.pallas import tpu_sc as plsc

N = 50176
R = 768
K = 12544
NSEG = 32
ZMAX = 5.5
SEGSCALE = NSEG / (2.0 * ZMAX)
CH = 1792
NCHUNK = N // CH
NW = 32
ROWS_PER_W = R // NW

_zb = np.linspace(-ZMAX, ZMAX, NSEG + 1)
_phi = np.array([0.5 * (1.0 + math.erf(z / math.sqrt(2.0))) for z in _zb])
_yb = _phi * (K - 2) + 1.0
_SLOPE = ((_yb[1:] - _yb[:-1]) / (_zb[1:] - _zb[:-1])).astype(np.float32)
_INTER = (_yb[:-1] - _SLOPE * _zb[:-1]).astype(np.float32)


def _stats_body(x_ref, o_ref):
    xb = x_ref[...]
    s = jnp.sum(xb)
    s2 = jnp.sum(xb * xb)
    col = lax.broadcasted_iota(jnp.int32, (1, 1, 128), 2)
    o_ref[...] = jnp.where(col == 0, s, jnp.where(col == 1, s2, 0.0))


def _gather_vec(v, idx):
    dn = lax.GatherDimensionNumbers(
        offset_dims=(), collapsed_slice_dims=(0,), start_index_map=(0,))
    return lax.gather(v, idx[:, None], dn, (1,),
                      mode=lax.GatherScatterMode.PROMISE_IN_BOUNDS)


def _sc_body(x_hbm, n_hbm, c_hbm, ls_hbm, li_hbm, out_hbm, vs_hbm,
             ga, gb, fx, fn, sa, sb, cv, lsv, liv):
    pltpu.sync_copy(c_hbm, cv)
    pltpu.sync_copy(ls_hbm, lsv)
    pltpu.sync_copy(li_hbm, liv)

    iota = lax.iota(jnp.int32, 16)
    mu_x = cv[0, :]
    is_x = cv[1, :]
    mu_n = cv[2, :]
    is_n = cv[3, :]
    av = cv[4, :]
    bv = cv[5, :]

    def bucketize(v, mu, inv_s):
        z = (v - mu) * inv_s
        seg = jnp.clip(((z + ZMAX) * SEGSCALE).astype(jnp.int32), 0, NSEG - 1)
        sl = plsc.load_gather(lsv, [seg])
        ic = plsc.load_gather(liv, [seg])
        return jnp.clip((sl * z + ic).astype(jnp.int32), 0, K - 1)

    wid = lax.axis_index("s") * 2 + lax.axis_index("c")

    def row_body(t, carry):
        r = wid * ROWS_PER_W + t
        sbase = wid * N

        def zb_(i, _):
            fx[pl.ds(i * 16, 16)] = jnp.zeros((16,), jnp.int32)
            fn[pl.ds(i * 16, 16)] = jnp.zeros((16,), jnp.int32)
            return 0
        lax.fori_loop(0, (K + 16) // 16, zb_, 0)

        def p1_chunk(c, _):
            pltpu.sync_copy(x_hbm.at[pl.ds(r * N + c * CH, CH)], sa)
            pltpu.sync_copy(n_hbm.at[pl.ds(r * N + c * CH, CH)], sb)

            def p1_v(ii, _):
                for u in range(4):
                    i = ii * 4 + u
                    xv = sa[pl.ds(i * 16, 16)]
                    nv = sb[pl.ds(i * 16, 16)]
                    nsv = av + bv * nv + 0.9 * xv
                    px = bucketize(xv, mu_x, is_x)
                    pn = bucketize(nsv, mu_n, is_n)
                    cntx, lastx = plsc.scan_count(px)
                    plsc.addupdate_scatter(fx, [px], cntx, mask=lastx)
                    cntn, lastn = plsc.scan_count(pn)
                    plsc.addupdate_scatter(fn, [pn], cntn, mask=lastn)
                return 0
            lax.fori_loop(0, CH // 64, p1_v, 0)
            return 0
        with jax.named_scope("ph_hist"):
            lax.fori_loop(0, NCHUNK, p1_chunk, 0)

        def excl(h_ref):
            def body(i, carry_v):
                v = h_ref[pl.ds(i * 16, 16)]
                inc = plsc.cumsum(v)
                h_ref[pl.ds(i * 16, 16)] = inc - v + carry_v
                return _gather_vec(inc, jnp.full((16,), 15, jnp.int32)) + carry_v
            lax.fori_loop(0, K // 16, body, jnp.zeros((16,), jnp.int32))
        with jax.named_scope("ph_cumsum"):
            excl(fx)
            excl(fn)

        def p2n_chunk(c, _):
            pltpu.sync_copy(x_hbm.at[pl.ds(r * N + c * CH, CH)], sa)
            pltpu.sync_copy(n_hbm.at[pl.ds(r * N + c * CH, CH)], sb)

            def p2n_v(ii, _):
                for u in range(4):
                    i = ii * 4 + u
                    xv = sa[pl.ds(i * 16, 16)]
                    nv = sb[pl.ds(i * 16, 16)]
                    nsv = av + bv * nv + 0.9 * xv
                    pn = bucketize(nsv, mu_n, is_n)
                    cntn, lastn = plsc.scan_count(pn)
                    basen = plsc.load_gather(fn, [pn])
                    plsc.store_scatter(ga, [basen + cntn - 1], nsv)
                    plsc.store_scatter(fn, [pn], basen + cntn, mask=lastn)
                return 0
            lax.fori_loop(0, CH // 64, p2n_v, 0)
            return 0
        with jax.named_scope("ph_scatter_n"):
            lax.fori_loop(0, NCHUNK, p2n_chunk, 0)
        ga[pl.ds(N, 16)] = jnp.full((16,), jnp.inf, jnp.float32)

        def blksort(ii, _):
            for u in range(4):
                i = ii * 4 + u
                v = ga[pl.ds(i * 16, 16)]
                sv, _u = plsc.sort_key_val(v, v)
                ga[pl.ds(i * 16, 16)] = sv
            return 0
        with jax.named_scope("ph_blksort_n"):
            lax.fori_loop(0, N // 64, blksort, 0)

        def merge_sweep(phase):
            def body(tt, acc):
                for u in range(2):
                    t2 = tt * 2 + u
                    base = (t2 * 2 + phase) * 16
                    a = ga[pl.ds(base, 16)]
                    b = ga[pl.ds(base + 16, 16)]
                    rb = lax.rev(b, (0,))
                    lo = jnp.minimum(a, rb)
                    hi = jnp.maximum(a, rb)
                    slo, _u1 = plsc.sort_key_val(lo, lo)
                    shi, _u2 = plsc.sort_key_val(hi, hi)
                    ga[pl.ds(base, 16)] = slo
                    ga[pl.ds(base + 16, 16)] = shi
                    ch = jnp.logical_or(slo != a, shi != b)
                    acc = jnp.logical_or(acc, ch)
                return acc
            return lax.fori_loop(0, N // 64, body, jnp.zeros((16,), jnp.bool_))

        def w_cond(c):
            return c

        def w_body(c):
            s0 = merge_sweep(0)
            s1 = merge_sweep(1)
            return jnp.any(jnp.logical_or(s0, s1))
        with jax.named_scope("ph_merge_n"):
            lax.while_loop(w_cond, w_body, jnp.bool_(True))

        with jax.named_scope("ph_stash"):
            pltpu.sync_copy(ga.at[pl.ds(0, N)], vs_hbm.at[pl.ds(sbase, N)])

        def p2x_chunk(c, _):
            pltpu.sync_copy(x_hbm.at[pl.ds(r * N + c * CH, CH)], sa)

            def p2x_v(ii, _):
                for u in range(4):
                    i = ii * 4 + u
                    xv = sa[pl.ds(i * 16, 16)]
                    idxf = (c * CH + i * 16 + iota).astype(jnp.float32)
                    px = bucketize(xv, mu_x, is_x)
                    cnt, last = plsc.scan_count(px)
                    base = plsc.load_gather(fx, [px])
                    dest = base + cnt - 1
                    plsc.store_scatter(gb, [dest], xv)
                    plsc.store_scatter(ga, [dest], idxf)
                    plsc.store_scatter(fx, [px], base + cnt, mask=last)
                return 0
            lax.fori_loop(0, CH // 64, p2x_v, 0)
            return 0
        with jax.named_scope("ph_scatter_x"):
            lax.fori_loop(0, NCHUNK, p2x_chunk, 0)
        gb[pl.ds(N, 16)] = jnp.full((16,), jnp.inf, jnp.float32)
        ga[pl.ds(N, 16)] = jnp.zeros((16,), jnp.float32)

        def blksort_p(ii, _):
            for u in range(4):
                i = ii * 4 + u
                kv = gb[pl.ds(i * 16, 16)]
                pv = ga[pl.ds(i * 16, 16)]
                sk, sp = plsc.sort_key_val(kv, pv)
                gb[pl.ds(i * 16, 16)] = sk
                ga[pl.ds(i * 16, 16)] = sp
            return 0
        with jax.named_scope("ph_blksort_x"):
            lax.fori_loop(0, N // 64, blksort_p, 0)

        def merge_sweep_p(phase):
            def body(tt, acc):
                for u in range(2):
                    t2 = tt * 2 + u
                    base = (t2 * 2 + phase) * 16
                    ak = gb[pl.ds(base, 16)]
                    ap = ga[pl.ds(base, 16)]
                    bk = gb[pl.ds(base + 16, 16)]
                    bp = ga[pl.ds(base + 16, 16)]
                    rbk = lax.rev(bk, (0,))
                    rbp = lax.rev(bp, (0,))
                    take = ak <= rbk
                    lok = jnp.where(take, ak, rbk)
                    lop = jnp.where(take, ap, rbp)
                    hik = jnp.where(take, rbk, ak)
                    hip = jnp.where(take, rbp, ap)
                    slok, slop = plsc.sort_key_val(lok, lop)
                    shik, ship = plsc.sort_key_val(hik, hip)
                    gb[pl.ds(base, 16)] = slok
                    ga[pl.ds(base, 16)] = slop
                    gb[pl.ds(base + 16, 16)] = shik
                    ga[pl.ds(base + 16, 16)] = ship
                    ch = jnp.logical_or(slok != ak, shik != bk)
                    acc = jnp.logical_or(acc, ch)
                return acc
            return lax.fori_loop(0, N // 64, body, jnp.zeros((16,), jnp.bool_))

        def wp_body(c):
            s0 = merge_sweep_p(0)
            s1 = merge_sweep_p(1)
            return jnp.any(jnp.logical_or(s0, s1))
        with jax.named_scope("ph_merge_x"):
            lax.while_loop(w_cond, wp_body, jnp.bool_(True))

        def fin_chunk(c, _):
            pltpu.sync_copy(vs_hbm.at[pl.ds(sbase + c * CH, CH)], sa)

            def fin_v(ii, _):
                for u in range(4):
                    i = ii * 4 + u
                    v = sa[pl.ds(i * 16, 16)]
                    iv = ga[pl.ds(c * CH + i * 16, 16)].astype(jnp.int32)
                    plsc.store_scatter(gb, [iv], v)
                return 0
            lax.fori_loop(0, CH // 64, fin_v, 0)
            return 0
        with jax.named_scope("ph_final"):
            lax.fori_loop(0, NCHUNK, fin_chunk, 0)
            pltpu.sync_copy(gb.at[pl.ds(0, N)], out_hbm.at[pl.ds(r * N, N)])
        return carry
    lax.fori_loop(0, ROWS_PER_W, row_body, 0)


_sc_call = pl.kernel(
    _sc_body,
    mesh=plsc.VectorSubcoreMesh(core_axis_name="c", subcore_axis_name="s"),
    compiler_params=pltpu.CompilerParams(needs_layout_passes=False),
    out_type=(jax.ShapeDtypeStruct((R * N,), jnp.float32),
              jax.ShapeDtypeStruct((NW * N,), jnp.float32)),
    scratch_types=[
        pltpu.VMEM((N + 16,), jnp.float32),
        pltpu.VMEM((N + 16,), jnp.float32),
        pltpu.VMEM((K + 16,), jnp.int32),
        pltpu.VMEM((K + 16,), jnp.int32),
        pltpu.VMEM((CH,), jnp.float32),
        pltpu.VMEM((CH,), jnp.float32),
        pltpu.VMEM((8, 16), jnp.float32),
        pltpu.VMEM((NSEG,), jnp.float32),
        pltpu.VMEM((NSEG,), jnp.float32),
    ],
)


def kernel(x, noise):
    B, C, W, H = x.shape
    x2 = x.reshape(R, N)
    n2 = noise.reshape(R, N)

    part = pl.pallas_call(
        _stats_body,
        grid=(96,),
        in_specs=[pl.BlockSpec((8, N), lambda i: (i, 0))],
        out_specs=pl.BlockSpec((1, 1, 128), lambda i: (i, 0, 0)),
        out_shape=jax.ShapeDtypeStruct((96, 1, 128), jnp.float32),
    )(x2)
    s = jnp.sum(part[:, 0, 0])
    s2 = jnp.sum(part[:, 0, 1])
    m = R * N
    mean = s / m
    var = (s2 - s * s / m) / (m - 1)
    sd = jnp.sqrt(var)

    def bc(v):
        return jnp.broadcast_to(v, (16,)).astype(jnp.float32)
    consts = jnp.stack([
        bc(mean), bc(1.0 / sd), bc(mean), bc(1.0 / (0.9055 * sd)),
        bc(0.1 * mean), bc(0.1 * sd), bc(0.0), bc(0.0)])

    out, _ = _sc_call(x2.reshape(-1), n2.reshape(-1), consts,
                      jnp.asarray(_SLOPE), jnp.asarray(_INTER))
    return out.reshape(B, C, W, H)


# speedup vs baseline: 6.4231x; 1.0750x over previous
import math

import numpy as np
import jax
import jax.numpy as jnp
from jax import lax
from jax.experimental import pallas as pl
from jax.experimental.pallas import tpu as pltpu
from jax.experimental.pallas import tpu_sc as plsc

N = 50176
R = 768
K = 12544
NSEG = 32
ZMAX = 5.5
SEGSCALE = NSEG / (2.0 * ZMAX)
CH = 1792
NCHUNK = N // CH
NW = 32
ROWS_PER_W = R // NW

_zb = np.linspace(-ZMAX, ZMAX, NSEG + 1)
_phi = np.array([0.5 * (1.0 + math.erf(z / math.sqrt(2.0))) for z in _zb])
_yb = _phi * (K - 2) + 1.0
_SLOPE = ((_yb[1:] - _yb[:-1]) / (_zb[1:] - _zb[:-1])).astype(np.float32)
_INTER = (_yb[:-1] - _SLOPE * _zb[:-1]).astype(np.float32)


def _stats_body(x_ref, o_ref):
    xb = x_ref[...]
    s = jnp.sum(xb)
    s2 = jnp.sum(xb * xb)
    col = lax.broadcasted_iota(jnp.int32, (1, 1, 128), 2)
    o_ref[...] = jnp.where(col == 0, s, jnp.where(col == 1, s2, 0.0))


def _gather_vec(v, idx):
    dn = lax.GatherDimensionNumbers(
        offset_dims=(), collapsed_slice_dims=(0,), start_index_map=(0,))
    return lax.gather(v, idx[:, None], dn, (1,),
                      mode=lax.GatherScatterMode.PROMISE_IN_BOUNDS)


def _sc_body(x_hbm, n_hbm, c_hbm, ls_hbm, li_hbm, out_hbm, vs_hbm,
             ga, gb, fx, fn, sa, sb, cv, lsv, liv):
    pltpu.sync_copy(c_hbm, cv)
    pltpu.sync_copy(ls_hbm, lsv)
    pltpu.sync_copy(li_hbm, liv)

    iota = lax.iota(jnp.int32, 16)
    mu_x = cv[0, :]
    is_x = cv[1, :]
    mu_n = cv[2, :]
    is_n = cv[3, :]
    av = cv[4, :]
    bv = cv[5, :]

    def bucketize(v, mu, inv_s):
        z = (v - mu) * inv_s
        seg = jnp.clip(((z + ZMAX) * SEGSCALE).astype(jnp.int32), 0, NSEG - 1)
        sl = plsc.load_gather(lsv, [seg])
        ic = plsc.load_gather(liv, [seg])
        return jnp.clip((sl * z + ic).astype(jnp.int32), 0, K - 1)

    wid = lax.axis_index("s") * 2 + lax.axis_index("c")

    def row_body(t, carry):
        r = wid * ROWS_PER_W + t
        sbase = wid * N

        def zb_(i, _):
            fx[pl.ds(i * 16, 16)] = jnp.zeros((16,), jnp.int32)
            fn[pl.ds(i * 16, 16)] = jnp.zeros((16,), jnp.int32)
            return 0
        lax.fori_loop(0, (K + 16) // 16, zb_, 0)

        def p1_chunk(c, _):
            pltpu.sync_copy(x_hbm.at[pl.ds(r * N + c * CH, CH)], sa)
            pltpu.sync_copy(n_hbm.at[pl.ds(r * N + c * CH, CH)], sb)

            def p1_v(ii, _):
                for u in range(4):
                    i = ii * 4 + u
                    xv = sa[pl.ds(i * 16, 16)]
                    nv = sb[pl.ds(i * 16, 16)]
                    nsv = av + bv * nv + 0.9 * xv
                    px = bucketize(xv, mu_x, is_x)
                    pn = bucketize(nsv, mu_n, is_n)
                    cntx, lastx = plsc.scan_count(px)
                    plsc.addupdate_scatter(fx, [px], cntx, mask=lastx)
                    cntn, lastn = plsc.scan_count(pn)
                    plsc.addupdate_scatter(fn, [pn], cntn, mask=lastn)
                    gb[pl.ds(c * CH + i * 16, 16)] = nsv
                return 0
            lax.fori_loop(0, CH // 64, p1_v, 0)
            return 0
        with jax.named_scope("ph_hist"):
            lax.fori_loop(0, NCHUNK, p1_chunk, 0)

        def excl(h_ref):
            def body(i, carry_v):
                v = h_ref[pl.ds(i * 16, 16)]
                inc = plsc.cumsum(v)
                h_ref[pl.ds(i * 16, 16)] = inc - v + carry_v
                return _gather_vec(inc, jnp.full((16,), 15, jnp.int32)) + carry_v
            lax.fori_loop(0, K // 16, body, jnp.zeros((16,), jnp.int32))
        with jax.named_scope("ph_cumsum"):
            excl(fx)
            excl(fn)

        def p2n_v(ii, _):
            for u in range(4):
                i = ii * 4 + u
                nsv = gb[pl.ds(i * 16, 16)]
                pn = bucketize(nsv, mu_n, is_n)
                cntn, lastn = plsc.scan_count(pn)
                basen = plsc.load_gather(fn, [pn])
                plsc.store_scatter(ga, [basen + cntn - 1], nsv)
                plsc.store_scatter(fn, [pn], basen + cntn, mask=lastn)
            return 0
        with jax.named_scope("ph_scatter_n"):
            lax.fori_loop(0, N // 64, p2n_v, 0)
        ga[pl.ds(N, 16)] = jnp.full((16,), jnp.inf, jnp.float32)

        def blksort(ii, _):
            for u in range(4):
                i = ii * 4 + u
                v = ga[pl.ds(i * 16, 16)]
                sv, _u = plsc.sort_key_val(v, v)
                ga[pl.ds(i * 16, 16)] = sv
            return 0
        with jax.named_scope("ph_blksort_n"):
            lax.fori_loop(0, N // 64, blksort, 0)

        def merge_sweep(phase):
            def body(tt, acc):
                for u in range(2):
                    t2 = tt * 2 + u
                    base = (t2 * 2 + phase) * 16
                    a = ga[pl.ds(base, 16)]
                    b = ga[pl.ds(base + 16, 16)]
                    rb = lax.rev(b, (0,))
                    lo = jnp.minimum(a, rb)
                    hi = jnp.maximum(a, rb)
                    slo, _u1 = plsc.sort_key_val(lo, lo)
                    shi, _u2 = plsc.sort_key_val(hi, hi)
                    ga[pl.ds(base, 16)] = slo
                    ga[pl.ds(base + 16, 16)] = shi
                    ch = jnp.logical_or(slo != a, shi != b)
                    acc = jnp.logical_or(acc, ch)
                return acc
            return lax.fori_loop(0, N // 64, body, jnp.zeros((16,), jnp.bool_))

        def w_cond(c):
            return c

        def w_body(c):
            s0 = merge_sweep(0)
            s1 = merge_sweep(1)
            return jnp.any(jnp.logical_or(s0, s1))
        with jax.named_scope("ph_merge_n"):
            lax.while_loop(w_cond, w_body, jnp.bool_(True))

        with jax.named_scope("ph_stash"):
            pltpu.sync_copy(ga.at[pl.ds(0, N)], vs_hbm.at[pl.ds(sbase, N)])

        def p2x_chunk(c, _):
            pltpu.sync_copy(x_hbm.at[pl.ds(r * N + c * CH, CH)], sa)

            def p2x_v(ii, _):
                for u in range(4):
                    i = ii * 4 + u
                    xv = sa[pl.ds(i * 16, 16)]
                    idxf = (c * CH + i * 16 + iota).astype(jnp.float32)
                    px = bucketize(xv, mu_x, is_x)
                    cnt, last = plsc.scan_count(px)
                    base = plsc.load_gather(fx, [px])
                    dest = base + cnt - 1
                    plsc.store_scatter(gb, [dest], xv)
                    plsc.store_scatter(ga, [dest], idxf)
                    plsc.store_scatter(fx, [px], base + cnt, mask=last)
                return 0
            lax.fori_loop(0, CH // 64, p2x_v, 0)
            return 0
        with jax.named_scope("ph_scatter_x"):
            lax.fori_loop(0, NCHUNK, p2x_chunk, 0)
        gb[pl.ds(N, 16)] = jnp.full((16,), jnp.inf, jnp.float32)
        ga[pl.ds(N, 16)] = jnp.zeros((16,), jnp.float32)

        def blksort_p(ii, _):
            for u in range(4):
                i = ii * 4 + u
                kv = gb[pl.ds(i * 16, 16)]
                pv = ga[pl.ds(i * 16, 16)]
                sk, sp = plsc.sort_key_val(kv, pv)
                gb[pl.ds(i * 16, 16)] = sk
                ga[pl.ds(i * 16, 16)] = sp
            return 0
        with jax.named_scope("ph_blksort_x"):
            lax.fori_loop(0, N // 64, blksort_p, 0)

        def merge_sweep_p(phase):
            def body(tt, acc):
                for u in range(2):
                    t2 = tt * 2 + u
                    base = (t2 * 2 + phase) * 16
                    ak = gb[pl.ds(base, 16)]
                    ap = ga[pl.ds(base, 16)]
                    bk = gb[pl.ds(base + 16, 16)]
                    bp = ga[pl.ds(base + 16, 16)]
                    rbk = lax.rev(bk, (0,))
                    rbp = lax.rev(bp, (0,))
                    take = ak <= rbk
                    lok = jnp.where(take, ak, rbk)
                    lop = jnp.where(take, ap, rbp)
                    hik = jnp.where(take, rbk, ak)
                    hip = jnp.where(take, rbp, ap)
                    slok, slop = plsc.sort_key_val(lok, lop)
                    shik, ship = plsc.sort_key_val(hik, hip)
                    gb[pl.ds(base, 16)] = slok
                    ga[pl.ds(base, 16)] = slop
                    gb[pl.ds(base + 16, 16)] = shik
                    ga[pl.ds(base + 16, 16)] = ship
                    ch = jnp.logical_or(slok != ak, shik != bk)
                    acc = jnp.logical_or(acc, ch)
                return acc
            return lax.fori_loop(0, N // 64, body, jnp.zeros((16,), jnp.bool_))

        def wp_body(c):
            s0 = merge_sweep_p(0)
            s1 = merge_sweep_p(1)
            return jnp.any(jnp.logical_or(s0, s1))
        with jax.named_scope("ph_merge_x"):
            lax.while_loop(w_cond, wp_body, jnp.bool_(True))

        def fin_chunk(c, _):
            pltpu.sync_copy(vs_hbm.at[pl.ds(sbase + c * CH, CH)], sa)

            def fin_v(ii, _):
                for u in range(4):
                    i = ii * 4 + u
                    v = sa[pl.ds(i * 16, 16)]
                    iv = ga[pl.ds(c * CH + i * 16, 16)].astype(jnp.int32)
                    plsc.store_scatter(gb, [iv], v)
                return 0
            lax.fori_loop(0, CH // 64, fin_v, 0)
            return 0
        with jax.named_scope("ph_final"):
            lax.fori_loop(0, NCHUNK, fin_chunk, 0)
            pltpu.sync_copy(gb.at[pl.ds(0, N)], out_hbm.at[pl.ds(r * N, N)])
        return carry
    lax.fori_loop(0, ROWS_PER_W, row_body, 0)


_sc_call = pl.kernel(
    _sc_body,
    mesh=plsc.VectorSubcoreMesh(core_axis_name="c", subcore_axis_name="s"),
    compiler_params=pltpu.CompilerParams(needs_layout_passes=False),
    out_type=(jax.ShapeDtypeStruct((R * N,), jnp.float32),
              jax.ShapeDtypeStruct((NW * N,), jnp.float32)),
    scratch_types=[
        pltpu.VMEM((N + 16,), jnp.float32),
        pltpu.VMEM((N + 16,), jnp.float32),
        pltpu.VMEM((K + 16,), jnp.int32),
        pltpu.VMEM((K + 16,), jnp.int32),
        pltpu.VMEM((CH,), jnp.float32),
        pltpu.VMEM((CH,), jnp.float32),
        pltpu.VMEM((8, 16), jnp.float32),
        pltpu.VMEM((NSEG,), jnp.float32),
        pltpu.VMEM((NSEG,), jnp.float32),
    ],
)


def kernel(x, noise):
    B, C, W, H = x.shape
    x2 = x.reshape(R, N)
    n2 = noise.reshape(R, N)

    part = pl.pallas_call(
        _stats_body,
        grid=(96,),
        in_specs=[pl.BlockSpec((8, N), lambda i: (i, 0))],
        out_specs=pl.BlockSpec((1, 1, 128), lambda i: (i, 0, 0)),
        out_shape=jax.ShapeDtypeStruct((96, 1, 128), jnp.float32),
    )(x2)
    s = jnp.sum(part[:, 0, 0])
    s2 = jnp.sum(part[:, 0, 1])
    m = R * N
    mean = s / m
    var = (s2 - s * s / m) / (m - 1)
    sd = jnp.sqrt(var)

    def bc(v):
        return jnp.broadcast_to(v, (16,)).astype(jnp.float32)
    consts = jnp.stack([
        bc(mean), bc(1.0 / sd), bc(mean), bc(1.0 / (0.9055 * sd)),
        bc(0.1 * mean), bc(0.1 * sd), bc(0.0), bc(0.0)])

    out, _ = _sc_call(x2.reshape(-1), n2.reshape(-1), consts,
                      jnp.asarray(_SLOPE), jnp.asarray(_INTER))
    return out.reshape(B, C, W, H)


# speedup vs baseline: 7.0281x; 1.0942x over previous
import math

import numpy as np
import jax
import jax.numpy as jnp
from jax import lax
from jax.experimental import pallas as pl
from jax.experimental.pallas import tpu as pltpu
from jax.experimental.pallas import tpu_sc as plsc

N = 50176
R = 768
K = 12544
NSEG = 32
ZMAX = 5.5
SEGSCALE = NSEG / (2.0 * ZMAX)
CH = 1792
NCHUNK = N // CH
NW = 32
ROWS_PER_W = R // NW

_zb = np.linspace(-ZMAX, ZMAX, NSEG + 1)
_phi = np.array([0.5 * (1.0 + math.erf(z / math.sqrt(2.0))) for z in _zb])
_yb = _phi * (K - 2) + 1.0
_SLOPE = ((_yb[1:] - _yb[:-1]) / (_zb[1:] - _zb[:-1])).astype(np.float32)
_INTER = (_yb[:-1] - _SLOPE * _zb[:-1]).astype(np.float32)


def _stats_body(x_ref, o_ref):
    xb = x_ref[...]
    s = jnp.sum(xb)
    s2 = jnp.sum(xb * xb)
    col = lax.broadcasted_iota(jnp.int32, (1, 1, 128), 2)
    o_ref[...] = jnp.where(col == 0, s, jnp.where(col == 1, s2, 0.0))


def _gather_vec(v, idx):
    dn = lax.GatherDimensionNumbers(
        offset_dims=(), collapsed_slice_dims=(0,), start_index_map=(0,))
    return lax.gather(v, idx[:, None], dn, (1,),
                      mode=lax.GatherScatterMode.PROMISE_IN_BOUNDS)


def _sc_body(x_hbm, n_hbm, c_hbm, ls_hbm, li_hbm, out_hbm, vs_hbm,
             ga, gb, fx, fn, sa, sb, cv, lsv, liv):
    pltpu.sync_copy(c_hbm, cv)
    pltpu.sync_copy(ls_hbm, lsv)
    pltpu.sync_copy(li_hbm, liv)

    iota = lax.iota(jnp.int32, 16)
    mu_x = cv[0, :]
    is_x = cv[1, :]
    mu_n = cv[2, :]
    is_n = cv[3, :]
    av = cv[4, :]
    bv = cv[5, :]

    def bucketize(v, mu, inv_s):
        z = (v - mu) * inv_s
        seg = jnp.clip(((z + ZMAX) * SEGSCALE).astype(jnp.int32), 0, NSEG - 1)
        sl = plsc.load_gather(lsv, [seg])
        ic = plsc.load_gather(liv, [seg])
        return jnp.clip((sl * z + ic).astype(jnp.int32), 0, K - 1)

    wid = lax.axis_index("s") * 2 + lax.axis_index("c")

    def row_body(t, carry):
        r = wid * ROWS_PER_W + t
        sbase = wid * N

        def zb_(i, _):
            fx[pl.ds(i * 16, 16)] = jnp.zeros((16,), jnp.int32)
            fn[pl.ds(i * 16, 16)] = jnp.zeros((16,), jnp.int32)
            return 0
        lax.fori_loop(0, (K + 16) // 16, zb_, 0)

        def p1_chunk(c, _):
            pltpu.sync_copy(x_hbm.at[pl.ds(r * N + c * CH, CH)], sa)
            pltpu.sync_copy(n_hbm.at[pl.ds(r * N + c * CH, CH)], sb)

            def p1_v(ii, _):
                for u in range(4):
                    i = ii * 4 + u
                    xv = sa[pl.ds(i * 16, 16)]
                    nv = sb[pl.ds(i * 16, 16)]
                    nsv = av + bv * nv + 0.9 * xv
                    px = bucketize(xv, mu_x, is_x)
                    pn = bucketize(nsv, mu_n, is_n)
                    cntx, lastx = plsc.scan_count(px)
                    plsc.addupdate_scatter(fx, [px], cntx, mask=lastx)
                    cntn, lastn = plsc.scan_count(pn)
                    plsc.addupdate_scatter(fn, [pn], cntn, mask=lastn)
                    gb[pl.ds(c * CH + i * 16, 16)] = nsv
                return 0
            lax.fori_loop(0, CH // 64, p1_v, 0)
            return 0
        with jax.named_scope("ph_hist"):
            lax.fori_loop(0, NCHUNK, p1_chunk, 0)

        lane15 = jnp.full((16,), 15, jnp.int32)

        def excl2(i, carry_v):
            cvx, cvn = carry_v
            vx_ = fx[pl.ds(i * 16, 16)]
            vn_ = fn[pl.ds(i * 16, 16)]
            incx = plsc.cumsum(vx_)
            incn = plsc.cumsum(vn_)
            fx[pl.ds(i * 16, 16)] = incx - vx_ + cvx
            fn[pl.ds(i * 16, 16)] = incn - vn_ + cvn
            return (_gather_vec(incx, lane15) + cvx,
                    _gather_vec(incn, lane15) + cvn)
        with jax.named_scope("ph_cumsum"):
            lax.fori_loop(0, K // 16, excl2,
                          (jnp.zeros((16,), jnp.int32),
                           jnp.zeros((16,), jnp.int32)))

        def p2n_v(ii, _):
            for u in range(4):
                i = ii * 4 + u
                nsv = gb[pl.ds(i * 16, 16)]
                pn = bucketize(nsv, mu_n, is_n)
                cntn, lastn = plsc.scan_count(pn)
                basen = plsc.load_gather(fn, [pn])
                plsc.store_scatter(ga, [basen + cntn - 1], nsv)
                plsc.store_scatter(fn, [pn], basen + cntn, mask=lastn)
            return 0
        with jax.named_scope("ph_scatter_n"):
            lax.fori_loop(0, N // 64, p2n_v, 0)
        ga[pl.ds(N, 16)] = jnp.full((16,), jnp.inf, jnp.float32)

        def blksort(ii, _):
            for u in range(4):
                i = ii * 4 + u
                v = ga[pl.ds(i * 16, 16)]
                sv, _u = plsc.sort_key_val(v, v)
                ga[pl.ds(i * 16, 16)] = sv
            return 0
        with jax.named_scope("ph_blksort_n"):
            lax.fori_loop(0, N // 64, blksort, 0)

        def merge_sweep(phase):
            def body(tt, acc):
                for u in range(4):
                    t2 = tt * 4 + u
                    base = (t2 * 2 + phase) * 16
                    a = ga[pl.ds(base, 16)]
                    b = ga[pl.ds(base + 16, 16)]
                    rb = lax.rev(b, (0,))
                    lo = jnp.minimum(a, rb)
                    hi = jnp.maximum(a, rb)
                    slo, _u1 = plsc.sort_key_val(lo, lo)
                    shi, _u2 = plsc.sort_key_val(hi, hi)
                    ga[pl.ds(base, 16)] = slo
                    ga[pl.ds(base + 16, 16)] = shi
                    ch = jnp.logical_or(slo != a, shi != b)
                    acc = jnp.logical_or(acc, ch)
                return acc
            return lax.fori_loop(0, N // 128, body, jnp.zeros((16,), jnp.bool_))

        def w_cond(c):
            return c

        def w_body(c):
            s0 = merge_sweep(0)
            s1 = merge_sweep(1)
            return jnp.any(jnp.logical_or(s0, s1))
        with jax.named_scope("ph_merge_n"):
            lax.while_loop(w_cond, w_body, jnp.bool_(True))

        with jax.named_scope("ph_stash"):
            pltpu.sync_copy(ga.at[pl.ds(0, N)], vs_hbm.at[pl.ds(sbase, N)])

        def p2x_chunk(c, _):
            pltpu.sync_copy(x_hbm.at[pl.ds(r * N + c * CH, CH)], sa)

            def p2x_v(ii, _):
                for u in range(4):
                    i = ii * 4 + u
                    xv = sa[pl.ds(i * 16, 16)]
                    idxf = (c * CH + i * 16 + iota).astype(jnp.float32)
                    px = bucketize(xv, mu_x, is_x)
                    cnt, last = plsc.scan_count(px)
                    base = plsc.load_gather(fx, [px])
                    dest = base + cnt - 1
                    plsc.store_scatter(gb, [dest], xv)
                    plsc.store_scatter(ga, [dest], idxf)
                    plsc.store_scatter(fx, [px], base + cnt, mask=last)
                return 0
            lax.fori_loop(0, CH // 64, p2x_v, 0)
            return 0
        with jax.named_scope("ph_scatter_x"):
            lax.fori_loop(0, NCHUNK, p2x_chunk, 0)
        gb[pl.ds(N, 16)] = jnp.full((16,), jnp.inf, jnp.float32)
        ga[pl.ds(N, 16)] = jnp.zeros((16,), jnp.float32)

        def blksort_p(ii, _):
            for u in range(4):
                i = ii * 4 + u
                kv = gb[pl.ds(i * 16, 16)]
                pv = ga[pl.ds(i * 16, 16)]
                sk, sp = plsc.sort_key_val(kv, pv)
                gb[pl.ds(i * 16, 16)] = sk
                ga[pl.ds(i * 16, 16)] = sp
            return 0
        with jax.named_scope("ph_blksort_x"):
            lax.fori_loop(0, N // 64, blksort_p, 0)

        def merge_sweep_p(phase):
            def body(tt, acc):
                for u in range(4):
                    t2 = tt * 4 + u
                    base = (t2 * 2 + phase) * 16
                    ak = gb[pl.ds(base, 16)]
                    ap = ga[pl.ds(base, 16)]
                    bk = gb[pl.ds(base + 16, 16)]
                    bp = ga[pl.ds(base + 16, 16)]
                    rbk = lax.rev(bk, (0,))
                    rbp = lax.rev(bp, (0,))
                    take = ak <= rbk
                    lok = jnp.where(take, ak, rbk)
                    lop = jnp.where(take, ap, rbp)
                    hik = jnp.where(take, rbk, ak)
                    hip = jnp.where(take, rbp, ap)
                    slok, slop = plsc.sort_key_val(lok, lop)
                    shik, ship = plsc.sort_key_val(hik, hip)
                    gb[pl.ds(base, 16)] = slok
                    ga[pl.ds(base, 16)] = slop
                    gb[pl.ds(base + 16, 16)] = shik
                    ga[pl.ds(base + 16, 16)] = ship
                    ch = jnp.logical_or(slok != ak, shik != bk)
                    acc = jnp.logical_or(acc, ch)
                return acc
            return lax.fori_loop(0, N // 128, body, jnp.zeros((16,), jnp.bool_))

        def wp_body(c):
            s0 = merge_sweep_p(0)
            s1 = merge_sweep_p(1)
            return jnp.any(jnp.logical_or(s0, s1))
        with jax.named_scope("ph_merge_x"):
            lax.while_loop(w_cond, wp_body, jnp.bool_(True))

        def fin_chunk(c, _):
            pltpu.sync_copy(vs_hbm.at[pl.ds(sbase + c * CH, CH)], sa)

            def fin_v(ii, _):
                for u in range(4):
                    i = ii * 4 + u
                    v = sa[pl.ds(i * 16, 16)]
                    iv = ga[pl.ds(c * CH + i * 16, 16)].astype(jnp.int32)
                    plsc.store_scatter(gb, [iv], v)
                return 0
            lax.fori_loop(0, CH // 64, fin_v, 0)
            return 0
        with jax.named_scope("ph_final"):
            lax.fori_loop(0, NCHUNK, fin_chunk, 0)
            pltpu.sync_copy(gb.at[pl.ds(0, N)], out_hbm.at[pl.ds(r * N, N)])
        return carry
    lax.fori_loop(0, ROWS_PER_W, row_body, 0)


_sc_call = pl.kernel(
    _sc_body,
    mesh=plsc.VectorSubcoreMesh(core_axis_name="c", subcore_axis_name="s"),
    compiler_params=pltpu.CompilerParams(needs_layout_passes=False),
    out_type=(jax.ShapeDtypeStruct((R * N,), jnp.float32),
              jax.ShapeDtypeStruct((NW * N,), jnp.float32)),
    scratch_types=[
        pltpu.VMEM((N + 16,), jnp.float32),
        pltpu.VMEM((N + 16,), jnp.float32),
        pltpu.VMEM((K + 16,), jnp.int32),
        pltpu.VMEM((K + 16,), jnp.int32),
        pltpu.VMEM((CH,), jnp.float32),
        pltpu.VMEM((CH,), jnp.float32),
        pltpu.VMEM((8, 16), jnp.float32),
        pltpu.VMEM((NSEG,), jnp.float32),
        pltpu.VMEM((NSEG,), jnp.float32),
    ],
)


def kernel(x, noise):
    B, C, W, H = x.shape
    x2 = x.reshape(R, N)
    n2 = noise.reshape(R, N)

    part = pl.pallas_call(
        _stats_body,
        grid=(96,),
        in_specs=[pl.BlockSpec((8, N), lambda i: (i, 0))],
        out_specs=pl.BlockSpec((1, 1, 128), lambda i: (i, 0, 0)),
        out_shape=jax.ShapeDtypeStruct((96, 1, 128), jnp.float32),
    )(x2)
    s = jnp.sum(part[:, 0, 0])
    s2 = jnp.sum(part[:, 0, 1])
    m = R * N
    mean = s / m
    var = (s2 - s * s / m) / (m - 1)
    sd = jnp.sqrt(var)

    def bc(v):
        return jnp.broadcast_to(v, (16,)).astype(jnp.float32)
    consts = jnp.stack([
        bc(mean), bc(1.0 / sd), bc(mean), bc(1.0 / (0.9055 * sd)),
        bc(0.1 * mean), bc(0.1 * sd), bc(0.0), bc(0.0)])

    out, _ = _sc_call(x2.reshape(-1), n2.reshape(-1), consts,
                      jnp.asarray(_SLOPE), jnp.asarray(_INTER))
    return out.reshape(B, C, W, H)


# speedup vs baseline: 7.2147x; 1.0266x over previous
import math

import numpy as np
import jax
import jax.numpy as jnp
from jax import lax
from jax.experimental import pallas as pl
from jax.experimental.pallas import tpu as pltpu
from jax.experimental.pallas import tpu_sc as plsc

N = 50176
R = 768
K = 12544
NSEG = 32
ZMAX = 5.5
SEGSCALE = NSEG / (2.0 * ZMAX)
CH = 1792
NCHUNK = N // CH
NW = 32
ROWS_PER_W = R // NW

_zb = np.linspace(-ZMAX, ZMAX, NSEG + 1)
_phi = np.array([0.5 * (1.0 + math.erf(z / math.sqrt(2.0))) for z in _zb])
_yb = _phi * (K - 2) + 1.0
_SLOPE = ((_yb[1:] - _yb[:-1]) / (_zb[1:] - _zb[:-1])).astype(np.float32)
_INTER = (_yb[:-1] - _SLOPE * _zb[:-1]).astype(np.float32)


def _stats_body(x_ref, o_ref):
    xb = x_ref[...]
    s = jnp.sum(xb)
    s2 = jnp.sum(xb * xb)
    col = lax.broadcasted_iota(jnp.int32, (1, 1, 128), 2)
    o_ref[...] = jnp.where(col == 0, s, jnp.where(col == 1, s2, 0.0))


def _gather_vec(v, idx):
    dn = lax.GatherDimensionNumbers(
        offset_dims=(), collapsed_slice_dims=(0,), start_index_map=(0,))
    return lax.gather(v, idx[:, None], dn, (1,),
                      mode=lax.GatherScatterMode.PROMISE_IN_BOUNDS)


def _sc_body(x_hbm, n_hbm, c_hbm, ls_hbm, li_hbm, out_hbm, vs_hbm,
             ga, gb, fx, fn, sa, sb, cv, lsv, liv):
    pltpu.sync_copy(c_hbm, cv)
    pltpu.sync_copy(ls_hbm, lsv)
    pltpu.sync_copy(li_hbm, liv)

    iota = lax.iota(jnp.int32, 16)
    mu_x = cv[0, :]
    is_x = cv[1, :]
    mu_n = cv[2, :]
    is_n = cv[3, :]
    av = cv[4, :]
    bv = cv[5, :]

    def bucketize(v, mu, inv_s):
        z = (v - mu) * inv_s
        seg = jnp.clip(((z + ZMAX) * SEGSCALE).astype(jnp.int32), 0, NSEG - 1)
        sl = plsc.load_gather(lsv, [seg])
        ic = plsc.load_gather(liv, [seg])
        return jnp.clip((sl * z + ic).astype(jnp.int32), 0, K - 1)

    wid = lax.axis_index("s") * 2 + lax.axis_index("c")

    def row_body(t, carry):
        r = wid * ROWS_PER_W + t
        sbase = wid * N

        def zb_(ii, _):
            for u in range(4):
                i = ii * 4 + u
                fx[pl.ds(i * 16, 16)] = jnp.zeros((16,), jnp.int32)
                fn[pl.ds(i * 16, 16)] = jnp.zeros((16,), jnp.int32)
            return 0
        lax.fori_loop(0, K // 64, zb_, 0)

        def p1_chunk(c, _):
            pltpu.sync_copy(x_hbm.at[pl.ds(r * N + c * CH, CH)], sa)
            pltpu.sync_copy(n_hbm.at[pl.ds(r * N + c * CH, CH)], sb)

            def p1_v(ii, _):
                for u in range(4):
                    i = ii * 4 + u
                    xv = sa[pl.ds(i * 16, 16)]
                    nv = sb[pl.ds(i * 16, 16)]
                    nsv = av + bv * nv + 0.9 * xv
                    px = bucketize(xv, mu_x, is_x)
                    pn = bucketize(nsv, mu_n, is_n)
                    cntx, lastx = plsc.scan_count(px)
                    plsc.addupdate_scatter(fx, [px], cntx, mask=lastx)
                    cntn, lastn = plsc.scan_count(pn)
                    plsc.addupdate_scatter(fn, [pn], cntn, mask=lastn)
                    gb[pl.ds(c * CH + i * 16, 16)] = nsv
                return 0
            lax.fori_loop(0, CH // 64, p1_v, 0)
            return 0
        with jax.named_scope("ph_hist"):
            lax.fori_loop(0, NCHUNK, p1_chunk, 0)

        lane15 = jnp.full((16,), 15, jnp.int32)

        def excl2(i, carry_v):
            cvx, cvn = carry_v
            vx_ = fx[pl.ds(i * 16, 16)]
            vn_ = fn[pl.ds(i * 16, 16)]
            incx = plsc.cumsum(vx_)
            incn = plsc.cumsum(vn_)
            fx[pl.ds(i * 16, 16)] = incx - vx_ + cvx
            fn[pl.ds(i * 16, 16)] = incn - vn_ + cvn
            return (_gather_vec(incx, lane15) + cvx,
                    _gather_vec(incn, lane15) + cvn)
        with jax.named_scope("ph_cumsum"):
            lax.fori_loop(0, K // 16, excl2,
                          (jnp.zeros((16,), jnp.int32),
                           jnp.zeros((16,), jnp.int32)))

        def p2n_v(ii, _):
            for u in range(8):
                i = ii * 8 + u
                nsv = gb[pl.ds(i * 16, 16)]
                pn = bucketize(nsv, mu_n, is_n)
                cntn, lastn = plsc.scan_count(pn)
                basen = plsc.load_gather(fn, [pn])
                plsc.store_scatter(ga, [basen + cntn - 1], nsv)
                plsc.store_scatter(fn, [pn], basen + cntn, mask=lastn)
            return 0
        with jax.named_scope("ph_scatter_n"):
            lax.fori_loop(0, N // 128, p2n_v, 0)
        ga[pl.ds(N, 16)] = jnp.full((16,), jnp.inf, jnp.float32)

        def blksort(ii, _):
            for u in range(8):
                i = ii * 8 + u
                v = ga[pl.ds(i * 16, 16)]
                sv, _u = plsc.sort_key_val(v, v)
                ga[pl.ds(i * 16, 16)] = sv
            return 0
        with jax.named_scope("ph_blksort_n"):
            lax.fori_loop(0, N // 128, blksort, 0)

        def merge_sweep(phase):
            def body(tt, acc):
                for u in range(4):
                    t2 = tt * 4 + u
                    base = (t2 * 2 + phase) * 16
                    a = ga[pl.ds(base, 16)]
                    b = ga[pl.ds(base + 16, 16)]
                    rb = lax.rev(b, (0,))
                    lo = jnp.minimum(a, rb)
                    hi = jnp.maximum(a, rb)
                    slo, _u1 = plsc.sort_key_val(lo, lo)
                    shi, _u2 = plsc.sort_key_val(hi, hi)
                    ga[pl.ds(base, 16)] = slo
                    ga[pl.ds(base + 16, 16)] = shi
                    ch = jnp.logical_or(slo != a, shi != b)
                    acc = jnp.logical_or(acc, ch)
                return acc
            return lax.fori_loop(0, N // 128, body, jnp.zeros((16,), jnp.bool_))

        def w_cond(c):
            return c

        def w_body(c):
            s0 = merge_sweep(0)
            s1 = merge_sweep(1)
            return jnp.any(jnp.logical_or(s0, s1))
        with jax.named_scope("ph_merge_n"):
            lax.while_loop(w_cond, w_body, jnp.bool_(True))

        with jax.named_scope("ph_stash"):
            pltpu.sync_copy(ga.at[pl.ds(0, N)], vs_hbm.at[pl.ds(sbase, N)])

        def p2x_chunk(c, _):
            pltpu.sync_copy(x_hbm.at[pl.ds(r * N + c * CH, CH)], sa)

            def p2x_v(ii, _):
                for u in range(4):
                    i = ii * 4 + u
                    xv = sa[pl.ds(i * 16, 16)]
                    idxf = (c * CH + i * 16 + iota).astype(jnp.float32)
                    px = bucketize(xv, mu_x, is_x)
                    cnt, last = plsc.scan_count(px)
                    base = plsc.load_gather(fx, [px])
                    dest = base + cnt - 1
                    plsc.store_scatter(gb, [dest], xv)
                    plsc.store_scatter(ga, [dest], idxf)
                    plsc.store_scatter(fx, [px], base + cnt, mask=last)
                return 0
            lax.fori_loop(0, CH // 64, p2x_v, 0)
            return 0
        with jax.named_scope("ph_scatter_x"):
            lax.fori_loop(0, NCHUNK, p2x_chunk, 0)
        gb[pl.ds(N, 16)] = jnp.full((16,), jnp.inf, jnp.float32)
        ga[pl.ds(N, 16)] = jnp.zeros((16,), jnp.float32)

        def blksort_p(ii, _):
            for u in range(8):
                i = ii * 8 + u
                kv = gb[pl.ds(i * 16, 16)]
                pv = ga[pl.ds(i * 16, 16)]
                sk, sp = plsc.sort_key_val(kv, pv)
                gb[pl.ds(i * 16, 16)] = sk
                ga[pl.ds(i * 16, 16)] = sp
            return 0
        with jax.named_scope("ph_blksort_x"):
            lax.fori_loop(0, N // 128, blksort_p, 0)

        def merge_sweep_p(phase):
            def body(tt, acc):
                for u in range(4):
                    t2 = tt * 4 + u
                    base = (t2 * 2 + phase) * 16
                    ak = gb[pl.ds(base, 16)]
                    ap = ga[pl.ds(base, 16)]
                    bk = gb[pl.ds(base + 16, 16)]
                    bp = ga[pl.ds(base + 16, 16)]
                    rbk = lax.rev(bk, (0,))
                    rbp = lax.rev(bp, (0,))
                    take = ak <= rbk
                    lok = jnp.where(take, ak, rbk)
                    lop = jnp.where(take, ap, rbp)
                    hik = jnp.where(take, rbk, ak)
                    hip = jnp.where(take, rbp, ap)
                    slok, slop = plsc.sort_key_val(lok, lop)
                    shik, ship = plsc.sort_key_val(hik, hip)
                    gb[pl.ds(base, 16)] = slok
                    ga[pl.ds(base, 16)] = slop
                    gb[pl.ds(base + 16, 16)] = shik
                    ga[pl.ds(base + 16, 16)] = ship
                    ch = jnp.logical_or(slok != ak, shik != bk)
                    acc = jnp.logical_or(acc, ch)
                return acc
            return lax.fori_loop(0, N // 128, body, jnp.zeros((16,), jnp.bool_))

        def wp_body(c):
            s0 = merge_sweep_p(0)
            s1 = merge_sweep_p(1)
            return jnp.any(jnp.logical_or(s0, s1))
        with jax.named_scope("ph_merge_x"):
            lax.while_loop(w_cond, wp_body, jnp.bool_(True))

        def fin_chunk(c, _):
            pltpu.sync_copy(vs_hbm.at[pl.ds(sbase + c * CH, CH)], sa)

            def fin_v(ii, _):
                for u in range(8):
                    i = ii * 8 + u
                    v = sa[pl.ds(i * 16, 16)]
                    iv = ga[pl.ds(c * CH + i * 16, 16)].astype(jnp.int32)
                    plsc.store_scatter(gb, [iv], v)
                return 0
            lax.fori_loop(0, CH // 128, fin_v, 0)
            return 0
        with jax.named_scope("ph_final"):
            lax.fori_loop(0, NCHUNK, fin_chunk, 0)
            pltpu.sync_copy(gb.at[pl.ds(0, N)], out_hbm.at[pl.ds(r * N, N)])
        return carry
    lax.fori_loop(0, ROWS_PER_W, row_body, 0)


_sc_call = pl.kernel(
    _sc_body,
    mesh=plsc.VectorSubcoreMesh(core_axis_name="c", subcore_axis_name="s"),
    compiler_params=pltpu.CompilerParams(needs_layout_passes=False),
    out_type=(jax.ShapeDtypeStruct((R * N,), jnp.float32),
              jax.ShapeDtypeStruct((NW * N,), jnp.float32)),
    scratch_types=[
        pltpu.VMEM((N + 16,), jnp.float32),
        pltpu.VMEM((N + 16,), jnp.float32),
        pltpu.VMEM((K + 16,), jnp.int32),
        pltpu.VMEM((K + 16,), jnp.int32),
        pltpu.VMEM((CH,), jnp.float32),
        pltpu.VMEM((CH,), jnp.float32),
        pltpu.VMEM((8, 16), jnp.float32),
        pltpu.VMEM((NSEG,), jnp.float32),
        pltpu.VMEM((NSEG,), jnp.float32),
    ],
)


def kernel(x, noise):
    B, C, W, H = x.shape
    x2 = x.reshape(R, N)
    n2 = noise.reshape(R, N)

    part = pl.pallas_call(
        _stats_body,
        grid=(96,),
        in_specs=[pl.BlockSpec((8, N), lambda i: (i, 0))],
        out_specs=pl.BlockSpec((1, 1, 128), lambda i: (i, 0, 0)),
        out_shape=jax.ShapeDtypeStruct((96, 1, 128), jnp.float32),
    )(x2)
    s = jnp.sum(part[:, 0, 0])
    s2 = jnp.sum(part[:, 0, 1])
    m = R * N
    mean = s / m
    var = (s2 - s * s / m) / (m - 1)
    sd = jnp.sqrt(var)

    def bc(v):
        return jnp.broadcast_to(v, (16,)).astype(jnp.float32)
    consts = jnp.stack([
        bc(mean), bc(1.0 / sd), bc(mean), bc(1.0 / (0.9055 * sd)),
        bc(0.1 * mean), bc(0.1 * sd), bc(0.0), bc(0.0)])

    out, _ = _sc_call(x2.reshape(-1), n2.reshape(-1), consts,
                      jnp.asarray(_SLOPE), jnp.asarray(_INTER))
    return out.reshape(B, C, W, H)


# speedup vs baseline: 7.2155x; 1.0001x over previous
import math

import numpy as np
import jax
import jax.numpy as jnp
from jax import lax
from jax.experimental import pallas as pl
from jax.experimental.pallas import tpu as pltpu
from jax.experimental.pallas import tpu_sc as plsc

N = 50176
R = 768
K = 12544
NSEG = 32
ZMAX = 5.5
SEGSCALE = NSEG / (2.0 * ZMAX)
CH = 1792
NCHUNK = N // CH
NW = 32
ROWS_PER_W = R // NW

_zb = np.linspace(-ZMAX, ZMAX, NSEG + 1)
_phi = np.array([0.5 * (1.0 + math.erf(z / math.sqrt(2.0))) for z in _zb])
_yb = _phi * (K - 2) + 1.0
_SLOPE = ((_yb[1:] - _yb[:-1]) / (_zb[1:] - _zb[:-1])).astype(np.float32)
_INTER = (_yb[:-1] - _SLOPE * _zb[:-1]).astype(np.float32)


def _stats_body(x_ref, o_ref):
    xb = x_ref[...]
    s = jnp.sum(xb)
    s2 = jnp.sum(xb * xb)
    col = lax.broadcasted_iota(jnp.int32, (1, 1, 128), 2)
    o_ref[...] = jnp.where(col == 0, s, jnp.where(col == 1, s2, 0.0))


def _gather_vec(v, idx):
    dn = lax.GatherDimensionNumbers(
        offset_dims=(), collapsed_slice_dims=(0,), start_index_map=(0,))
    return lax.gather(v, idx[:, None], dn, (1,),
                      mode=lax.GatherScatterMode.PROMISE_IN_BOUNDS)


def _sc_body(x_hbm, n_hbm, c_hbm, ls_hbm, li_hbm, out_hbm, vs_hbm,
             ga, gb, fx, fn, sa, sb, cv, lsv, liv):
    pltpu.sync_copy(c_hbm, cv)
    pltpu.sync_copy(ls_hbm, lsv)
    pltpu.sync_copy(li_hbm, liv)

    iota = lax.iota(jnp.int32, 16)
    mu_x = cv[0, :]
    is_x = cv[1, :]
    mu_n = cv[2, :]
    is_n = cv[3, :]
    av = cv[4, :]
    bv = cv[5, :]

    def bucketize(v, mu, inv_s):
        z = (v - mu) * inv_s
        seg = jnp.clip(((z + ZMAX) * SEGSCALE).astype(jnp.int32), 0, NSEG - 1)
        sl = plsc.load_gather(lsv, [seg])
        ic = plsc.load_gather(liv, [seg])
        return jnp.clip((sl * z + ic).astype(jnp.int32), 0, K - 1)

    wid = lax.axis_index("s") * 2 + lax.axis_index("c")

    def row_body(t, carry):
        r = wid * ROWS_PER_W + t
        sbase = wid * N

        def zb_(ii, _):
            for u in range(4):
                i = ii * 4 + u
                fx[pl.ds(i * 16, 16)] = jnp.zeros((16,), jnp.int32)
                fn[pl.ds(i * 16, 16)] = jnp.zeros((16,), jnp.int32)
            return 0
        lax.fori_loop(0, K // 64, zb_, 0)

        def p1_chunk(c, _):
            pltpu.sync_copy(x_hbm.at[pl.ds(r * N + c * CH, CH)], sa)
            pltpu.sync_copy(n_hbm.at[pl.ds(r * N + c * CH, CH)], sb)

            def p1_v(ii, _):
                for u in range(8):
                    i = ii * 8 + u
                    xv = sa[pl.ds(i * 16, 16)]
                    nv = sb[pl.ds(i * 16, 16)]
                    nsv = av + bv * nv + 0.9 * xv
                    px = bucketize(xv, mu_x, is_x)
                    pn = bucketize(nsv, mu_n, is_n)
                    cntx, lastx = plsc.scan_count(px)
                    plsc.addupdate_scatter(fx, [px], cntx, mask=lastx)
                    cntn, lastn = plsc.scan_count(pn)
                    plsc.addupdate_scatter(fn, [pn], cntn, mask=lastn)
                    gb[pl.ds(c * CH + i * 16, 16)] = nsv
                return 0
            lax.fori_loop(0, CH // 128, p1_v, 0)
            return 0
        with jax.named_scope("ph_hist"):
            lax.fori_loop(0, NCHUNK, p1_chunk, 0)

        lane15 = jnp.full((16,), 15, jnp.int32)

        def excl2(i, carry_v):
            cvx, cvn = carry_v
            vx_ = fx[pl.ds(i * 16, 16)]
            vn_ = fn[pl.ds(i * 16, 16)]
            incx = plsc.cumsum(vx_)
            incn = plsc.cumsum(vn_)
            fx[pl.ds(i * 16, 16)] = incx - vx_ + cvx
            fn[pl.ds(i * 16, 16)] = incn - vn_ + cvn
            return (_gather_vec(incx, lane15) + cvx,
                    _gather_vec(incn, lane15) + cvn)
        with jax.named_scope("ph_cumsum"):
            lax.fori_loop(0, K // 16, excl2,
                          (jnp.zeros((16,), jnp.int32),
                           jnp.zeros((16,), jnp.int32)))

        def p2n_v(ii, _):
            for u in range(8):
                i = ii * 8 + u
                nsv = gb[pl.ds(i * 16, 16)]
                pn = bucketize(nsv, mu_n, is_n)
                cntn, lastn = plsc.scan_count(pn)
                basen = plsc.load_gather(fn, [pn])
                plsc.store_scatter(ga, [basen + cntn - 1], nsv)
                plsc.store_scatter(fn, [pn], basen + cntn, mask=lastn)
            return 0
        with jax.named_scope("ph_scatter_n"):
            lax.fori_loop(0, N // 128, p2n_v, 0)
        ga[pl.ds(N, 16)] = jnp.full((16,), jnp.inf, jnp.float32)

        def blksort(ii, _):
            for u in range(8):
                i = ii * 8 + u
                v = ga[pl.ds(i * 16, 16)]
                sv, _u = plsc.sort_key_val(v, v)
                ga[pl.ds(i * 16, 16)] = sv
            return 0
        with jax.named_scope("ph_blksort_n"):
            lax.fori_loop(0, N // 128, blksort, 0)

        def merge_sweep(phase):
            def body(tt, acc):
                for u in range(4):
                    t2 = tt * 4 + u
                    base = (t2 * 2 + phase) * 16
                    a = ga[pl.ds(base, 16)]
                    b = ga[pl.ds(base + 16, 16)]
                    rb = lax.rev(b, (0,))
                    lo = jnp.minimum(a, rb)
                    hi = jnp.maximum(a, rb)
                    slo, _u1 = plsc.sort_key_val(lo, lo)
                    shi, _u2 = plsc.sort_key_val(hi, hi)
                    ga[pl.ds(base, 16)] = slo
                    ga[pl.ds(base + 16, 16)] = shi
                    ch = jnp.logical_or(slo != a, shi != b)
                    acc = jnp.logical_or(acc, ch)
                return acc
            return lax.fori_loop(0, N // 128, body, jnp.zeros((16,), jnp.bool_))

        def w_cond(c):
            return c

        def w_body(c):
            s0 = merge_sweep(0)
            s1 = merge_sweep(1)
            return jnp.any(jnp.logical_or(s0, s1))
        with jax.named_scope("ph_merge_n"):
            lax.while_loop(w_cond, w_body, jnp.bool_(True))

        with jax.named_scope("ph_stash"):
            pltpu.sync_copy(ga.at[pl.ds(0, N)], vs_hbm.at[pl.ds(sbase, N)])

        def p2x_chunk(c, _):
            pltpu.sync_copy(x_hbm.at[pl.ds(r * N + c * CH, CH)], sa)

            def p2x_v(ii, _):
                for u in range(8):
                    i = ii * 8 + u
                    xv = sa[pl.ds(i * 16, 16)]
                    idxf = (c * CH + i * 16 + iota).astype(jnp.float32)
                    px = bucketize(xv, mu_x, is_x)
                    cnt, last = plsc.scan_count(px)
                    base = plsc.load_gather(fx, [px])
                    dest = base + cnt - 1
                    plsc.store_scatter(gb, [dest], xv)
                    plsc.store_scatter(ga, [dest], idxf)
                    plsc.store_scatter(fx, [px], base + cnt, mask=last)
                return 0
            lax.fori_loop(0, CH // 128, p2x_v, 0)
            return 0
        with jax.named_scope("ph_scatter_x"):
            lax.fori_loop(0, NCHUNK, p2x_chunk, 0)
        gb[pl.ds(N, 16)] = jnp.full((16,), jnp.inf, jnp.float32)
        ga[pl.ds(N, 16)] = jnp.zeros((16,), jnp.float32)

        def blksort_p(ii, _):
            for u in range(8):
                i = ii * 8 + u
                kv = gb[pl.ds(i * 16, 16)]
                pv = ga[pl.ds(i * 16, 16)]
                sk, sp = plsc.sort_key_val(kv, pv)
                gb[pl.ds(i * 16, 16)] = sk
                ga[pl.ds(i * 16, 16)] = sp
            return 0
        with jax.named_scope("ph_blksort_x"):
            lax.fori_loop(0, N // 128, blksort_p, 0)

        def merge_sweep_p(phase):
            def body(tt, acc):
                for u in range(4):
                    t2 = tt * 4 + u
                    base = (t2 * 2 + phase) * 16
                    ak = gb[pl.ds(base, 16)]
                    ap = ga[pl.ds(base, 16)]
                    bk = gb[pl.ds(base + 16, 16)]
                    bp = ga[pl.ds(base + 16, 16)]
                    rbk = lax.rev(bk, (0,))
                    rbp = lax.rev(bp, (0,))
                    take = ak <= rbk
                    lok = jnp.where(take, ak, rbk)
                    lop = jnp.where(take, ap, rbp)
                    hik = jnp.where(take, rbk, ak)
                    hip = jnp.where(take, rbp, ap)
                    slok, slop = plsc.sort_key_val(lok, lop)
                    shik, ship = plsc.sort_key_val(hik, hip)
                    gb[pl.ds(base, 16)] = slok
                    ga[pl.ds(base, 16)] = slop
                    gb[pl.ds(base + 16, 16)] = shik
                    ga[pl.ds(base + 16, 16)] = ship
                    ch = jnp.logical_or(slok != ak, shik != bk)
                    acc = jnp.logical_or(acc, ch)
                return acc
            return lax.fori_loop(0, N // 128, body, jnp.zeros((16,), jnp.bool_))

        def wp_body(c):
            s0 = merge_sweep_p(0)
            s1 = merge_sweep_p(1)
            return jnp.any(jnp.logical_or(s0, s1))
        with jax.named_scope("ph_merge_x"):
            lax.while_loop(w_cond, wp_body, jnp.bool_(True))

        def fin_chunk(c, _):
            pltpu.sync_copy(vs_hbm.at[pl.ds(sbase + c * CH, CH)], sa)

            def fin_v(ii, _):
                for u in range(8):
                    i = ii * 8 + u
                    v = sa[pl.ds(i * 16, 16)]
                    iv = ga[pl.ds(c * CH + i * 16, 16)].astype(jnp.int32)
                    plsc.store_scatter(gb, [iv], v)
                return 0
            lax.fori_loop(0, CH // 128, fin_v, 0)
            return 0
        with jax.named_scope("ph_final"):
            lax.fori_loop(0, NCHUNK, fin_chunk, 0)
            pltpu.sync_copy(gb.at[pl.ds(0, N)], out_hbm.at[pl.ds(r * N, N)])
        return carry
    lax.fori_loop(0, ROWS_PER_W, row_body, 0)


_sc_call = pl.kernel(
    _sc_body,
    mesh=plsc.VectorSubcoreMesh(core_axis_name="c", subcore_axis_name="s"),
    compiler_params=pltpu.CompilerParams(needs_layout_passes=False),
    out_type=(jax.ShapeDtypeStruct((R * N,), jnp.float32),
              jax.ShapeDtypeStruct((NW * N,), jnp.float32)),
    scratch_types=[
        pltpu.VMEM((N + 16,), jnp.float32),
        pltpu.VMEM((N + 16,), jnp.float32),
        pltpu.VMEM((K + 16,), jnp.int32),
        pltpu.VMEM((K + 16,), jnp.int32),
        pltpu.VMEM((CH,), jnp.float32),
        pltpu.VMEM((CH,), jnp.float32),
        pltpu.VMEM((8, 16), jnp.float32),
        pltpu.VMEM((NSEG,), jnp.float32),
        pltpu.VMEM((NSEG,), jnp.float32),
    ],
)


def kernel(x, noise):
    B, C, W, H = x.shape
    x2 = x.reshape(R, N)
    n2 = noise.reshape(R, N)

    part = pl.pallas_call(
        _stats_body,
        grid=(96,),
        in_specs=[pl.BlockSpec((8, N), lambda i: (i, 0))],
        out_specs=pl.BlockSpec((1, 1, 128), lambda i: (i, 0, 0)),
        out_shape=jax.ShapeDtypeStruct((96, 1, 128), jnp.float32),
    )(x2)
    s = jnp.sum(part[:, 0, 0])
    s2 = jnp.sum(part[:, 0, 1])
    m = R * N
    mean = s / m
    var = (s2 - s * s / m) / (m - 1)
    sd = jnp.sqrt(var)

    def bc(v):
        return jnp.broadcast_to(v, (16,)).astype(jnp.float32)
    consts = jnp.stack([
        bc(mean), bc(1.0 / sd), bc(mean), bc(1.0 / (0.9055 * sd)),
        bc(0.1 * mean), bc(0.1 * sd), bc(0.0), bc(0.0)])

    out, _ = _sc_call(x2.reshape(-1), n2.reshape(-1), consts,
                      jnp.asarray(_SLOPE), jnp.asarray(_INTER))
    return out.reshape(B, C, W, H)
